# baseline (speedup 1.0000x reference)
import functools

import jax
import jax.numpy as jnp
from jax import lax
from jax.experimental import pallas as pl
from jax.experimental.pallas import tpu as pltpu

_NB = 4
_G = 16
_S = _NB * _G * _G
_R = 2048
_F32 = jnp.float32


def _gelu(x):
    return x * 0.5 * (1.0 + lax.erf(x * 0.7071067811865476))


def _mlpw(p):
    ws = []
    for layer in p:
        ws.append(layer["W"])
        ws.append(layer["b"].reshape(1, -1))
    return ws


def _mlp(x, ws):
    n = len(ws) // 2
    for i in range(n):
        x = jnp.dot(x, ws[2 * i], preferred_element_type=_F32) + ws[2 * i + 1]
        if i < n - 1:
            x = _gelu(x)
    return x


def _dotg0(a, b):
    return lax.dot_general(a, b, (((0,), (0,)), ((), ())),
                           preferred_element_type=_F32)


def _hilo(x):
    hi = x.astype(jnp.bfloat16)
    lo = (x - hi.astype(_F32)).astype(jnp.bfloat16)
    return hi, lo


def _scatter_oh(oh, vals):
    hi, lo = _hilo(vals)
    return _dotg0(oh, hi) + _dotg0(oh, lo)


def _gather_oh(oh, tab):
    hi, lo = _hilo(tab)
    return (jnp.dot(oh, hi, preferred_element_type=_F32) +
            jnp.dot(oh, lo, preferred_element_type=_F32))


def _onehot_rows(pos, batch):
    cx = jnp.clip(jnp.floor(pos[:, 0:1] * _G).astype(jnp.int32), 0, _G - 1)
    cy = jnp.clip(jnp.floor(pos[:, 1:2] * _G).astype(jnp.int32), 0, _G - 1)
    sub = batch * (_G * _G) + cx * _G + cy
    ids = lax.broadcasted_iota(jnp.int32, (pos.shape[0], _S), 1)
    return (sub == ids).astype(jnp.bfloat16)


def _bh(batch):
    ids = lax.broadcasted_iota(jnp.int32, (batch.shape[0], 8), 1)
    return (batch == ids).astype(_F32)


def _in_gelu(x, batch, bstats):
    C = x.shape[1]
    s = jnp.dot(_bh(batch), bstats, preferred_element_type=_F32)
    cnt = jnp.maximum(s[:, 2 * C:2 * C + 1], 1.0)
    mean = s[:, :C] / cnt
    var = s[:, C:2 * C] / cnt - mean * mean
    return _gelu((x - mean) / jnp.sqrt(var + 1e-5))


def _acc_init(ref):
    @pl.when(pl.program_id(0) == 0)
    def _():
        ref[...] = jnp.zeros_like(ref)


def _p1_body(x_ref, pos_ref, b_ref, *rest):
    ws = [r[...] for r in rest[:-2]]
    x0_ref, acc_ref = rest[-2], rest[-1]
    pos, b = pos_ref[...], b_ref[...]
    pe = _mlp(pos, ws[0:6])
    lf = _mlp(x_ref[...], ws[6:10])
    x0 = jnp.concatenate([pe, lf], axis=1)
    x0_ref[...] = x0
    v = _mlp(x0, ws[10:14])
    oh = _onehot_rows(pos, b)
    ones = jnp.ones((pos.shape[0], 1), _F32)
    vals = jnp.concatenate([v, pos, ones], axis=1)
    _acc_init(acc_ref)
    acc_ref[...] += _scatter_oh(oh, vals)


def _pg_body(pos_ref, b_ref, tab_ref, *rest, ccol):
    ws = [r[...] for r in rest[:-2]]
    out_ref, bst_ref = rest[-2], rest[-1]
    pos, b = pos_ref[...], b_ref[...]
    oh = _onehot_rows(pos, b)
    g = _gather_oh(oh, tab_ref[...])
    if ccol is None:
        mean = g
    else:
        mean = g[:, :32] / jnp.maximum(g[:, ccol:ccol + 1], 1.0)
    out = _mlp(jnp.concatenate([pos, mean], axis=1), ws)
    out_ref[...] = out
    ones = jnp.ones((pos.shape[0], 1), _F32)
    sb = jnp.concatenate([out, out * out, ones], axis=1)
    _acc_init(bst_ref)
    bst_ref[...] += _dotg0(_bh(b), sb)


def _ns_body(pos_ref, b_ref, prev_ref, bst_ref, *rest, nws, with_x0):
    k = 1 if with_x0 else 0
    x0 = rest[0][...] if with_x0 else None
    ws = [r[...] for r in rest[k:k + nws]]
    acc_ref = rest[-1]
    pos, b = pos_ref[...], b_ref[...]
    h = _in_gelu(prev_ref[...], b, bst_ref[...])
    if with_x0:
        h = jnp.concatenate([h, x0], axis=1)
    v = _mlp(h, ws) if nws else h
    oh = _onehot_rows(pos, b)
    ones = jnp.ones((pos.shape[0], 1), _F32)
    _acc_init(acc_ref)
    acc_ref[...] += _scatter_oh(oh, jnp.concatenate([v, ones], axis=1))


def _p11_body(b_ref, prev_ref, bst_ref, *rest):
    ws = [r[...] for r in rest[:-1]]
    y_ref = rest[-1]
    h = _in_gelu(prev_ref[...], b_ref[...], bst_ref[...])
    y_ref[...] = _mlp(h, ws)


def _mid_body(acc1_ref, acc3_ref, *rest):
    out_ref = rest[-1]
    loaded = iter([r[...] for r in rest[:-1]])

    def take4():
        return [next(loaded) for _ in range(4)]

    def ohm(ppos, pb, n):
        m = ppos.shape[0]
        cx = jnp.clip(jnp.floor(ppos[:, 0:1] * n).astype(jnp.int32), 0, n - 1)
        cy = jnp.clip(jnp.floor(ppos[:, 1:2] * n).astype(jnp.int32), 0, n - 1)
        sub = pb * (n * n) + cx * n + cy
        ids = lax.broadcasted_iota(jnp.int32, (m, _NB * n * n), 1)
        return (sub == ids).astype(_F32)

    def bhm(pb):
        ids = lax.broadcasted_iota(jnp.int32, (pb.shape[0], _NB), 1)
        return (pb == ids).astype(_F32)

    def dd(x, s_oh, t_oh, tpos, vws, tws):
        v = _mlp(x, vws)
        c = v.shape[1]
        ones = jnp.ones((x.shape[0], 1), _F32)
        sums = _dotg0(s_oh, jnp.concatenate([v, ones], axis=1))
        g = jnp.dot(t_oh, sums, preferred_element_type=_F32)
        mean = g[:, :c] / jnp.maximum(g[:, c:c + 1], 1.0)
        return _mlp(jnp.concatenate([tpos, mean], axis=1), tws)

    def inorm(x, bho):
        c = x.shape[1]
        ones = jnp.ones((x.shape[0], 1), _F32)
        s = _dotg0(bho, jnp.concatenate([x, x * x, ones], axis=1))
        row = jnp.dot(bho, s, preferred_element_type=_F32)
        cnt = jnp.maximum(row[:, 2 * c:2 * c + 1], 1.0)
        mean = row[:, :c] / cnt
        var = row[:, c:2 * c] / cnt - mean * mean
        return _gelu((x - mean) / jnp.sqrt(var + 1e-5))

    def blockf(x, spos, s_oh, s_bh, tpos, t_oh, t_bh):
        o = dd(x, s_oh, s_oh, spos, take4(), take4())
        o = inorm(o, s_bh)
        o = dd(o, s_oh, t_oh, tpos, take4(), take4())
        return inorm(o, t_bh)

    def pool(x, ppos, oh):
        ones = jnp.ones((x.shape[0], 1), _F32)
        ps = _dotg0(oh, jnp.concatenate([x, ppos, ones], axis=1))
        c = x.shape[1]
        cnt = jnp.maximum(ps[:, c + 2:c + 3], 1.0)
        return ps[:, :c] / cnt, ps[:, c:c + 2] / cnt

    acc1 = acc1_ref[...]
    acc3 = acc3_ref[...]
    cnt1 = jnp.maximum(acc1[:, 34:35], 1.0)
    p1pos = acc1[:, 32:34] / cnt1
    p1x = acc3[:, 0:32] / cnt1
    p1b = lax.broadcasted_iota(jnp.int32, (1024, 1), 0) // 256
    p2b = lax.broadcasted_iota(jnp.int32, (256, 1), 0) // 64
    p3b = lax.broadcasted_iota(jnp.int32, (64, 1), 0) // 16
    p4b = lax.broadcasted_iota(jnp.int32, (16, 1), 0) // 4
    bh1, bh2, bh3, bh4 = bhm(p1b), bhm(p2b), bhm(p3b), bhm(p4b)

    oh_p1_8 = ohm(p1pos, p1b, 8)
    e2 = blockf(p1x, p1pos, oh_p1_8, bh1, p1pos, oh_p1_8, bh1)
    p2x, p2pos = pool(e2, p1pos, oh_p1_8)
    oh_p2_4 = ohm(p2pos, p2b, 4)
    e3 = blockf(p2x, p2pos, oh_p2_4, bh2, p2pos, oh_p2_4, bh2)
    p3x, p3pos = pool(e3, p2pos, oh_p2_4)
    oh_p3_2 = ohm(p3pos, p3b, 2)
    e4 = blockf(p3x, p3pos, oh_p3_2, bh3, p3pos, oh_p3_2, bh3)
    p4x, p4pos = pool(e4, p3pos, oh_p3_2)
    oh_p4_1 = ohm(p4pos, p4b, 1)
    bb = blockf(p4x, p4pos, oh_p4_1, bh4, p4pos, oh_p4_1, bh4)
    bb = blockf(bb, p4pos, oh_p4_1, bh4, p4pos, oh_p4_1, bh4)
    oh_p4_2 = ohm(p4pos, p4b, 2)
    d4 = blockf(jnp.concatenate([bb, p4x], axis=1), p4pos, oh_p4_2,
                bh4, p3pos, oh_p3_2, bh3)
    oh_p3_4 = ohm(p3pos, p3b, 4)
    d3 = blockf(jnp.concatenate([d4, p3x], axis=1), p3pos, oh_p3_4,
                bh3, p2pos, oh_p2_4, bh2)
    oh_p2_8 = ohm(p2pos, p2b, 8)
    d2 = blockf(jnp.concatenate([d3, p2x], axis=1), p2pos, oh_p2_8,
                bh2, p1pos, oh_p1_8, bh1)
    oh_p1_16 = ohm(p1pos, p1b, 16)
    o = dd(jnp.concatenate([d2, p1x], axis=1), oh_p1_16, oh_p1_16,
           p1pos, take4(), take4())
    h = inorm(o, bh1)
    v = _mlp(h, take4())
    ones = jnp.ones((1024, 1), _F32)
    sums = _dotg0(oh_p1_16, jnp.concatenate([v, ones], axis=1))
    out_ref[...] = sums[:, :32] / jnp.maximum(sums[:, 32:33], 1.0)


def _rows(c):
    return pl.BlockSpec((_R, c), lambda i: (i, 0))


def _full(a):
    nd = a.ndim
    return pl.BlockSpec(a.shape, lambda i: (0,) * nd)


def _sds(shape):
    return jax.ShapeDtypeStruct(shape, _F32)


def kernel(x, pos, batch, params):
    n = x.shape[0]
    nt = -(-n // _R)
    npad = nt * _R
    padn = npad - n
    xp = jnp.pad(x, ((0, padn), (0, 0)))
    posp = jnp.pad(pos, ((0, padn), (0, 0)))
    bp = jnp.pad(batch.astype(jnp.int32), (0, padn),
                 constant_values=_NB).reshape(npad, 1)

    def call(body, ins, outs, out_specs):
        specs = []
        for a, kind in ins:
            specs.append(_rows(kind) if isinstance(kind, int) else _full(a))
        return pl.pallas_call(
            body,
            grid=(nt,),
            in_specs=specs,
            out_specs=out_specs,
            out_shape=outs,
        )(*[a for a, _ in ins])

    p = params
    e1o1, e1o2 = p["enc1"]["op1"], p["enc1"]["op2"]
    d0o1, d0o2 = p["dec0"]["op1"], p["dec0"]["op2"]

    ws1 = (_mlpw(p["point_encode"]) + _mlpw(p["lift"]) + _mlpw(e1o1["op_v"]))
    x0, acc1 = call(
        _p1_body,
        [(xp, 128), (posp, 2), (bp, 1)] + [(w, None) for w in ws1],
        [_sds((npad, 96)), _sds((_S, 35))],
        [_rows(96), _full(jnp.zeros((_S, 35)))],
    )

    def gather_pass(tab, tws, ccol):
        return call(
            functools.partial(_pg_body, ccol=ccol),
            [(posp, 2), (bp, 1), (tab, None)] + [(w, None) for w in tws],
            [_sds((npad, 32)), _sds((8, 65))],
            [_rows(32), _full(jnp.zeros((8, 65)))],
        )

    def ns_pass(prev, bst, ws, with_x0=False):
        ins = [(posp, 2), (bp, 1), (prev, 32), (bst, None)]
        if with_x0:
            ins.append((x0, 96))
        ins += [(w, None) for w in ws]
        return call(
            functools.partial(_ns_body, nws=len(ws), with_x0=with_x0),
            ins,
            _sds((_S, 33)),
            _full(jnp.zeros((_S, 33))),
        )

    out1, bst1 = gather_pass(acc1, _mlpw(e1o1["op_tgt_kernel"]), ccol=34)
    acc2 = ns_pass(out1, bst1, _mlpw(e1o2["op_v"]))
    out2, bst2 = gather_pass(acc2, _mlpw(e1o2["op_tgt_kernel"]), ccol=32)
    acc3 = ns_pass(out2, bst2, [])

    mid_ws = []
    for name in ["enc2", "enc3", "enc4", "bot1", "bot2",
                 "dec4", "dec3", "dec2"]:
        for opn in ["op1", "op2"]:
            mid_ws += _mlpw(p[name][opn]["op_v"])
            mid_ws += _mlpw(p[name][opn]["op_tgt_kernel"])
    mid_ws += _mlpw(p["dec1"]["op1"]["op_v"])
    mid_ws += _mlpw(p["dec1"]["op1"]["op_tgt_kernel"])
    mid_ws += _mlpw(p["dec1"]["op2"]["op_v"])
    d1tab = call(
        _mid_body,
        [(acc1, None), (acc3, None)] + [(w, None) for w in mid_ws],
        _sds((_S, 32)),
        _full(jnp.zeros((_S, 32))),
    )

    out_d1, bst3 = gather_pass(d1tab, _mlpw(p["dec1"]["op2"]["op_tgt_kernel"]),
                               ccol=None)
    acc4 = ns_pass(out_d1, bst3, _mlpw(d0o1["op_v"]), with_x0=True)
    out_e, bst4 = gather_pass(acc4, _mlpw(d0o1["op_tgt_kernel"]), ccol=32)
    acc5 = ns_pass(out_e, bst4, _mlpw(d0o2["op_v"]))
    out_f, bst5 = gather_pass(acc5, _mlpw(d0o2["op_tgt_kernel"]), ccol=32)

    y = call(
        _p11_body,
        [(bp, 1), (out_f, 32), (bst5, None)] + [(w, None)
                                                for w in _mlpw(p["project"])],
        _sds((npad, 128)),
        _rows(128),
    )
    return y[:n]


# speedup vs baseline: 1.1398x; 1.1398x over previous
import functools

import jax
import jax.numpy as jnp
from jax import lax
from jax.experimental import pallas as pl
from jax.experimental.pallas import tpu as pltpu
from jax.experimental.pallas import tpu_sc as plsc

_NB = 4
_G = 16
_S = _NB * _G * _G
_R = 2048
_F32 = jnp.float32

_NC, _NS = 2, 16
_NW = _NC * _NS
_CHUNK = 112
_ALIGN = _R * 7
_TR = 1152
_VC = 48


def _gelu(x):
    return x * 0.5 * (1.0 + lax.erf(x * 0.7071067811865476))


def _mlpw(p):
    ws = []
    for layer in p:
        ws.append(layer["W"])
        ws.append(layer["b"].reshape(1, -1))
    return ws


def _mlp(x, ws):
    n = len(ws) // 2
    for i in range(n):
        x = jnp.dot(x, ws[2 * i], preferred_element_type=_F32) + ws[2 * i + 1]
        if i < n - 1:
            x = _gelu(x)
    return x


def _dotg0(a, b):
    return lax.dot_general(a, b, (((0,), (0,)), ((), ())),
                           preferred_element_type=_F32)


def _subid(pos, batch):
    cx = jnp.clip(jnp.floor(pos[:, 0:1] * _G).astype(jnp.int32), 0, _G - 1)
    cy = jnp.clip(jnp.floor(pos[:, 1:2] * _G).astype(jnp.int32), 0, _G - 1)
    return batch * (_G * _G) + cx * _G + cy


def _bh(batch):
    ids = lax.broadcasted_iota(jnp.int32, (batch.shape[0], 8), 1)
    return (batch == ids).astype(_F32)


def _in_gelu(x, batch, bstats):
    C = x.shape[1]
    s = jnp.dot(_bh(batch), bstats, preferred_element_type=_F32)
    cnt = jnp.maximum(s[:, 2 * C:2 * C + 1], 1.0)
    mean = s[:, :C] / cnt
    var = s[:, C:2 * C] / cnt - mean * mean
    return _gelu((x - mean) / jnp.sqrt(var + 1e-5))


def _acc_init(ref):
    @pl.when(pl.program_id(0) == 0)
    def _():
        ref[...] = jnp.zeros_like(ref)


def _vals48(v, pos=None):
    r = v.shape[0]
    ones = jnp.ones((r, 1), _F32)
    mid = pos if pos is not None else jnp.zeros((r, 2), _F32)
    return jnp.concatenate([v, mid, ones, jnp.zeros((r, _VC - 35), _F32)],
                           axis=1)


def _p1_body(x_ref, pos_ref, b_ref, *rest):
    ws = [r[...] for r in rest[:-3]]
    x0_ref, vals_ref, sub_ref = rest[-3], rest[-2], rest[-1]
    pos, b = pos_ref[...], b_ref[...]
    pe = _mlp(pos, ws[0:6])
    lf = _mlp(x_ref[...], ws[6:10])
    x0 = jnp.concatenate([pe, lf], axis=1)
    x0_ref[...] = x0
    v = _mlp(x0, ws[10:14])
    vals_ref[...] = _vals48(v, pos)
    sub_ref[...] = _subid(pos, b)


def _g_body(pos_ref, b_ref, g_ref, *rest):
    ws = [r[...] for r in rest[:-2]]
    out_ref, bst_ref = rest[-2], rest[-1]
    pos, b = pos_ref[...], b_ref[...]
    out = _mlp(jnp.concatenate([pos, g_ref[...]], axis=1), ws)
    out_ref[...] = out
    ones = jnp.ones((pos.shape[0], 1), _F32)
    sb = jnp.concatenate([out, out * out, ones], axis=1)
    _acc_init(bst_ref)
    bst_ref[...] += _dotg0(_bh(b), sb)


def _ns_body(b_ref, prev_ref, bst_ref, *rest, nws, with_x0):
    k = 1 if with_x0 else 0
    x0 = rest[0][...] if with_x0 else None
    ws = [r[...] for r in rest[k:k + nws]]
    vals_ref = rest[-1]
    b = b_ref[...]
    h = _in_gelu(prev_ref[...], b, bst_ref[...])
    if with_x0:
        h = jnp.concatenate([h, x0], axis=1)
    v = _mlp(h, ws) if nws else h
    vals_ref[...] = _vals48(v)


def _p11_body(b_ref, prev_ref, bst_ref, *rest):
    ws = [r[...] for r in rest[:-1]]
    y_ref = rest[-1]
    h = _in_gelu(prev_ref[...], b_ref[...], bst_ref[...])
    y_ref[...] = _mlp(h, ws)


def _merge_body(p_ref, *outs, want_sums, want_mean):
    p = p_ref[0] + p_ref[1]
    i = 0
    if want_sums:
        outs[i][...] = p
        i += 1
    if want_mean:
        outs[i][...] = p[:, :32] / jnp.maximum(p[:, 34:35], 1.0)


def _sc_scatter(vals, idx2d, zeros, cpt):
    mesh = plsc.VectorSubcoreMesh(core_axis_name="c", subcore_axis_name="s",
                                  num_cores=_NC, num_subcores=_NS)
    rpt = _TR // _NS

    @functools.partial(
        pl.kernel,
        out_type=jax.ShapeDtypeStruct((_NC, _TR, _VC), _F32),
        mesh=mesh,
        scratch_types=[
            pltpu.VMEM_SHARED((_TR, _VC), _F32),
            pltpu.VMEM((cpt, _CHUNK), jnp.int32),
            pltpu.VMEM((_CHUNK, _VC), _F32),
        ],
        compiler_params=pltpu.CompilerParams(use_tc_tiling_on_sc=False),
    )
    def scat(vals_hbm, idx_hbm, zeros_hbm, out_hbm, table, idx_v, buf):
        cc = lax.axis_index("c")
        ss = lax.axis_index("s")
        wid = ss * _NC + cc
        pltpu.sync_copy(zeros_hbm.at[pl.ds(ss * rpt, rpt)],
                        table.at[pl.ds(ss * rpt, rpt)])
        pltpu.sync_copy(idx_hbm.at[wid], idx_v)
        plsc.subcore_barrier()

        def step(j, cr):
            off = (wid * cpt + j) * _CHUNK
            pltpu.sync_copy(vals_hbm.at[pl.ds(off, _CHUNK)], buf)
            pltpu.sync_copy(buf, table.at[idx_v.at[j]], add=True)
            return cr

        lax.fori_loop(0, cpt, step, 0)
        plsc.subcore_barrier()
        pltpu.sync_copy(table.at[pl.ds(ss * rpt, rpt)],
                        out_hbm.at[cc, pl.ds(ss * rpt, rpt)])

    return scat(vals, idx2d, zeros)


def _sc_gather(tab, idx2d, npad, cpt):
    mesh = plsc.VectorSubcoreMesh(core_axis_name="c", subcore_axis_name="s",
                                  num_cores=_NC, num_subcores=_NS)

    @functools.partial(
        pl.kernel,
        out_type=jax.ShapeDtypeStruct((npad, 32), _F32),
        mesh=mesh,
        scratch_types=[
            pltpu.VMEM((cpt, _CHUNK), jnp.int32),
            pltpu.VMEM((_CHUNK, 32), _F32),
            pltpu.SemaphoreType.DMA,
        ],
        compiler_params=pltpu.CompilerParams(use_tc_tiling_on_sc=False),
    )
    def gat(tab_hbm, idx_hbm, out_hbm, idx_v, buf, sem):
        cc = lax.axis_index("c")
        ss = lax.axis_index("s")
        wid = ss * _NC + cc
        pltpu.sync_copy(idx_hbm.at[wid], idx_v)

        def step(j, cr):
            off = (wid * cpt + j) * _CHUNK
            pltpu.async_copy(tab_hbm.at[idx_v.at[j]], buf, sem).wait()
            pltpu.sync_copy(buf, out_hbm.at[pl.ds(off, _CHUNK)])
            return cr

        lax.fori_loop(0, cpt, step, 0)

    return gat(tab, idx2d)


def _mid_body(m1_ref, m3_ref, *rest):
    out_ref = rest[-1]
    loaded = iter([r[...] for r in rest[:-1]])

    def take4():
        return [next(loaded) for _ in range(4)]

    def ohm(ppos, pb, n):
        m = ppos.shape[0]
        cx = jnp.clip(jnp.floor(ppos[:, 0:1] * n).astype(jnp.int32), 0, n - 1)
        cy = jnp.clip(jnp.floor(ppos[:, 1:2] * n).astype(jnp.int32), 0, n - 1)
        sub = pb * (n * n) + cx * n + cy
        ids = lax.broadcasted_iota(jnp.int32, (m, _NB * n * n), 1)
        return (sub == ids).astype(_F32)

    def bhm(pb):
        ids = lax.broadcasted_iota(jnp.int32, (pb.shape[0], _NB), 1)
        return (pb == ids).astype(_F32)

    def dd(x, s_oh, t_oh, tpos, vws, tws):
        v = _mlp(x, vws)
        c = v.shape[1]
        ones = jnp.ones((x.shape[0], 1), _F32)
        sums = _dotg0(s_oh, jnp.concatenate([v, ones], axis=1))
        g = jnp.dot(t_oh, sums, preferred_element_type=_F32)
        mean = g[:, :c] / jnp.maximum(g[:, c:c + 1], 1.0)
        return _mlp(jnp.concatenate([tpos, mean], axis=1), tws)

    def inorm(x, bho):
        c = x.shape[1]
        ones = jnp.ones((x.shape[0], 1), _F32)
        s = _dotg0(bho, jnp.concatenate([x, x * x, ones], axis=1))
        row = jnp.dot(bho, s, preferred_element_type=_F32)
        cnt = jnp.maximum(row[:, 2 * c:2 * c + 1], 1.0)
        mean = row[:, :c] / cnt
        var = row[:, c:2 * c] / cnt - mean * mean
        return _gelu((x - mean) / jnp.sqrt(var + 1e-5))

    def blockf(x, spos, s_oh, s_bh, tpos, t_oh, t_bh):
        o = dd(x, s_oh, s_oh, spos, take4(), take4())
        o = inorm(o, s_bh)
        o = dd(o, s_oh, t_oh, tpos, take4(), take4())
        return inorm(o, t_bh)

    def pool(x, ppos, oh):
        ones = jnp.ones((x.shape[0], 1), _F32)
        ps = _dotg0(oh, jnp.concatenate([x, ppos, ones], axis=1))
        c = x.shape[1]
        cnt = jnp.maximum(ps[:, c + 2:c + 3], 1.0)
        return ps[:, :c] / cnt, ps[:, c:c + 2] / cnt

    m1 = m1_ref[...]
    m3 = m3_ref[...]
    cnt1 = jnp.maximum(m1[:_S, 34:35], 1.0)
    p1pos = m1[:_S, 32:34] / cnt1
    p1x = m3[:_S, 0:32] / cnt1
    p1b = lax.broadcasted_iota(jnp.int32, (1024, 1), 0) // 256
    p2b = lax.broadcasted_iota(jnp.int32, (256, 1), 0) // 64
    p3b = lax.broadcasted_iota(jnp.int32, (64, 1), 0) // 16
    p4b = lax.broadcasted_iota(jnp.int32, (16, 1), 0) // 4
    bh1, bh2, bh3, bh4 = bhm(p1b), bhm(p2b), bhm(p3b), bhm(p4b)

    oh_p1_8 = ohm(p1pos, p1b, 8)
    e2 = blockf(p1x, p1pos, oh_p1_8, bh1, p1pos, oh_p1_8, bh1)
    p2x, p2pos = pool(e2, p1pos, oh_p1_8)
    oh_p2_4 = ohm(p2pos, p2b, 4)
    e3 = blockf(p2x, p2pos, oh_p2_4, bh2, p2pos, oh_p2_4, bh2)
    p3x, p3pos = pool(e3, p2pos, oh_p2_4)
    oh_p3_2 = ohm(p3pos, p3b, 2)
    e4 = blockf(p3x, p3pos, oh_p3_2, bh3, p3pos, oh_p3_2, bh3)
    p4x, p4pos = pool(e4, p3pos, oh_p3_2)
    oh_p4_1 = ohm(p4pos, p4b, 1)
    bb = blockf(p4x, p4pos, oh_p4_1, bh4, p4pos, oh_p4_1, bh4)
    bb = blockf(bb, p4pos, oh_p4_1, bh4, p4pos, oh_p4_1, bh4)
    oh_p4_2 = ohm(p4pos, p4b, 2)
    d4 = blockf(jnp.concatenate([bb, p4x], axis=1), p4pos, oh_p4_2,
                bh4, p3pos, oh_p3_2, bh3)
    oh_p3_4 = ohm(p3pos, p3b, 4)
    d3 = blockf(jnp.concatenate([d4, p3x], axis=1), p3pos, oh_p3_4,
                bh3, p2pos, oh_p2_4, bh2)
    oh_p2_8 = ohm(p2pos, p2b, 8)
    d2 = blockf(jnp.concatenate([d3, p2x], axis=1), p2pos, oh_p2_8,
                bh2, p1pos, oh_p1_8, bh1)
    oh_p1_16 = ohm(p1pos, p1b, 16)
    o = dd(jnp.concatenate([d2, p1x], axis=1), oh_p1_16, oh_p1_16,
           p1pos, take4(), take4())
    h = inorm(o, bh1)
    v = _mlp(h, take4())
    ones = jnp.ones((1024, 1), _F32)
    sums = _dotg0(oh_p1_16, jnp.concatenate([v, ones], axis=1))
    out_ref[...] = sums[:, :32] / jnp.maximum(sums[:, 32:33], 1.0)


def _rows(c):
    return pl.BlockSpec((_R, c), lambda i: (i, 0))


def _full(a):
    nd = a.ndim
    return pl.BlockSpec(a.shape, lambda i: (0,) * nd)


def _sds(shape, dt=_F32):
    return jax.ShapeDtypeStruct(shape, dt)


def kernel(x, pos, batch, params):
    n = x.shape[0]
    npad = -(-n // _ALIGN) * _ALIGN
    nt = npad // _R
    cpt = npad // (_NW * _CHUNK)
    padn = npad - n
    xp = jnp.pad(x, ((0, padn), (0, 0)))
    posp = jnp.pad(pos, ((0, padn), (0, 0)))
    bp = jnp.pad(batch.astype(jnp.int32), (0, padn),
                 constant_values=_NB).reshape(npad, 1)
    zeros_tab = jnp.zeros((_TR, _VC), _F32)

    def call(body, ins, outs, out_specs):
        specs = []
        for a, kind in ins:
            specs.append(_rows(kind) if isinstance(kind, int) else _full(a))
        return pl.pallas_call(
            body,
            grid=(nt,),
            in_specs=specs,
            out_specs=out_specs,
            out_shape=outs,
        )(*[a for a, _ in ins])

    def merge(parts, want_sums, want_mean):
        outs, specs = [], []
        if want_sums:
            outs.append(_sds((_TR, _VC)))
            specs.append(_full(jnp.zeros((_TR, _VC))))
        if want_mean:
            outs.append(_sds((_TR, 32)))
            specs.append(_full(jnp.zeros((_TR, 32))))
        r = pl.pallas_call(
            functools.partial(_merge_body, want_sums=want_sums,
                              want_mean=want_mean),
            grid=(1,),
            in_specs=[_full(parts)],
            out_specs=specs,
            out_shape=outs,
        )(parts)
        return r if len(r) > 1 else r[0]

    p = params
    e1o1, e1o2 = p["enc1"]["op1"], p["enc1"]["op2"]
    d0o1, d0o2 = p["dec0"]["op1"], p["dec0"]["op2"]

    ws1 = (_mlpw(p["point_encode"]) + _mlpw(p["lift"]) + _mlpw(e1o1["op_v"]))
    x0, vals1, sub = call(
        _p1_body,
        [(xp, 128), (posp, 2), (bp, 1)] + [(w, None) for w in ws1],
        [_sds((npad, 96)), _sds((npad, _VC)), _sds((npad, 1), jnp.int32)],
        [_rows(96), _rows(_VC), _rows(1)],
    )
    idx2d = sub.reshape(_NW, cpt, _CHUNK)

    def gather_pass(gat, tws):
        return call(
            _g_body,
            [(posp, 2), (bp, 1), (gat, 32)] + [(w, None) for w in tws],
            [_sds((npad, 32)), _sds((8, 65))],
            [_rows(32), _full(jnp.zeros((8, 65)))],
        )

    def ns_pass(prev, bst, ws, with_x0=False):
        ins = [(bp, 1), (prev, 32), (bst, None)]
        if with_x0:
            ins.append((x0, 96))
        ins += [(w, None) for w in ws]
        return call(
            functools.partial(_ns_body, nws=len(ws), with_x0=with_x0),
            ins,
            _sds((npad, _VC)),
            _rows(_VC),
        )

    def seg_roundtrip(vals, want_sums=False):
        parts = _sc_scatter(vals, idx2d, zeros_tab, cpt)
        m = merge(parts, want_sums, True)
        sums, mean = m if want_sums else (None, m)
        return sums, _sc_gather(mean, idx2d, npad, cpt)

    merged1, g1 = seg_roundtrip(vals1, want_sums=True)
    out1, bst1 = gather_pass(g1, _mlpw(e1o1["op_tgt_kernel"]))
    vals2 = ns_pass(out1, bst1, _mlpw(e1o2["op_v"]))
    _, g2 = seg_roundtrip(vals2)
    out2, bst2 = gather_pass(g2, _mlpw(e1o2["op_tgt_kernel"]))
    vals3 = ns_pass(out2, bst2, [])
    parts3 = _sc_scatter(vals3, idx2d, zeros_tab, cpt)
    merged3 = merge(parts3, True, False)

    mid_ws = []
    for name in ["enc2", "enc3", "enc4", "bot1", "bot2",
                 "dec4", "dec3", "dec2"]:
        for opn in ["op1", "op2"]:
            mid_ws += _mlpw(p[name][opn]["op_v"])
            mid_ws += _mlpw(p[name][opn]["op_tgt_kernel"])
    mid_ws += _mlpw(p["dec1"]["op1"]["op_v"])
    mid_ws += _mlpw(p["dec1"]["op1"]["op_tgt_kernel"])
    mid_ws += _mlpw(p["dec1"]["op2"]["op_v"])
    d1tab = call(
        _mid_body,
        [(merged1, None), (merged3, None)] + [(w, None) for w in mid_ws],
        _sds((_S, 32)),
        _full(jnp.zeros((_S, 32))),
    )
    d1pad = jnp.pad(d1tab, ((0, _TR - _S), (0, 0)))
    gmid = _sc_gather(d1pad, idx2d, npad, cpt)

    out_d1, bst3 = gather_pass(gmid, _mlpw(p["dec1"]["op2"]["op_tgt_kernel"]))
    vals4 = ns_pass(out_d1, bst3, _mlpw(d0o1["op_v"]), with_x0=True)
    _, g4 = seg_roundtrip(vals4)
    out_e, bst4 = gather_pass(g4, _mlpw(d0o1["op_tgt_kernel"]))
    vals5 = ns_pass(out_e, bst4, _mlpw(d0o2["op_v"]))
    _, g5 = seg_roundtrip(vals5)
    out_f, bst5 = gather_pass(g5, _mlpw(d0o2["op_tgt_kernel"]))

    y = call(
        _p11_body,
        [(bp, 1), (out_f, 32), (bst5, None)] + [(w, None)
                                                for w in _mlpw(p["project"])],
        _sds((npad, 128)),
        _rows(128),
    )
    return y[:n]


# speedup vs baseline: 1.7438x; 1.5299x over previous
import functools

import jax
import jax.numpy as jnp
from jax import lax
from jax.experimental import pallas as pl
from jax.experimental.pallas import tpu as pltpu
from jax.experimental.pallas import tpu_sc as plsc

_NB = 4
_G = 16
_S = _NB * _G * _G
_R = 7168
_F32 = jnp.float32

_NC, _NS = 2, 16
_NW = _NC * _NS
_CHUNK = 112
_ALIGN = _R
_TR = 1152
_VC = 48


def _gelu(x):
    return x * 0.5 * (1.0 + lax.erf(x * 0.7071067811865476))


def _mlpw(p):
    ws = []
    for layer in p:
        ws.append(layer["W"])
        ws.append(layer["b"].reshape(1, -1))
    return ws


def _mlp(x, ws):
    n = len(ws) // 2
    for i in range(n):
        x = jnp.dot(x, ws[2 * i], preferred_element_type=_F32) + ws[2 * i + 1]
        if i < n - 1:
            x = _gelu(x)
    return x


def _dotg0(a, b):
    return lax.dot_general(a, b, (((0,), (0,)), ((), ())),
                           preferred_element_type=_F32)


def _subid(pos, batch):
    cx = jnp.clip(jnp.floor(pos[:, 0:1] * _G).astype(jnp.int32), 0, _G - 1)
    cy = jnp.clip(jnp.floor(pos[:, 1:2] * _G).astype(jnp.int32), 0, _G - 1)
    return batch * (_G * _G) + cx * _G + cy


def _bh(batch):
    ids = lax.broadcasted_iota(jnp.int32, (batch.shape[0], 8), 1)
    return (batch == ids).astype(_F32)


def _in_gelu(x, batch, bstats):
    C = x.shape[1]
    s = jnp.dot(_bh(batch), bstats, preferred_element_type=_F32)
    cnt = jnp.maximum(s[:, 2 * C:2 * C + 1], 1.0)
    mean = s[:, :C] / cnt
    var = s[:, C:2 * C] / cnt - mean * mean
    return _gelu((x - mean) / jnp.sqrt(var + 1e-5))


def _acc_init(ref):
    @pl.when(pl.program_id(0) == 0)
    def _():
        ref[...] = jnp.zeros_like(ref)


def _vals48(v, pos=None):
    r = v.shape[0]
    ones = jnp.ones((r, 1), _F32)
    mid = pos if pos is not None else jnp.zeros((r, 2), _F32)
    return jnp.concatenate([v, mid, ones, jnp.zeros((r, _VC - 35), _F32)],
                           axis=1)


def _p1_body(x_ref, pos_ref, b_ref, *rest):
    ws = [r[...] for r in rest[:-3]]
    x0_ref, vals_ref, sub_ref = rest[-3], rest[-2], rest[-1]
    pos, b = pos_ref[...], b_ref[...]
    pe = _mlp(pos, ws[0:6])
    lf = _mlp(x_ref[...], ws[6:10])
    x0 = jnp.concatenate([pe, lf], axis=1)
    x0_ref[...] = x0
    v = _mlp(x0, ws[10:14])
    vals_ref[...] = _vals48(v, pos)
    sub_ref[...] = _subid(pos, b)


def _g_body(pos_ref, b_ref, g_ref, *rest):
    ws = [r[...] for r in rest[:-2]]
    out_ref, bst_ref = rest[-2], rest[-1]
    pos, b = pos_ref[...], b_ref[...]
    out = _mlp(jnp.concatenate([pos, g_ref[...]], axis=1), ws)
    out_ref[...] = out
    ones = jnp.ones((pos.shape[0], 1), _F32)
    sb = jnp.concatenate([out, out * out, ones], axis=1)
    _acc_init(bst_ref)
    bst_ref[...] += _dotg0(_bh(b), sb)


def _ns_body(b_ref, prev_ref, bst_ref, *rest, nws, with_x0):
    k = 1 if with_x0 else 0
    x0 = rest[0][...] if with_x0 else None
    ws = [r[...] for r in rest[k:k + nws]]
    vals_ref = rest[-1]
    b = b_ref[...]
    h = _in_gelu(prev_ref[...], b, bst_ref[...])
    if with_x0:
        h = jnp.concatenate([h, x0], axis=1)
    v = _mlp(h, ws) if nws else h
    vals_ref[...] = _vals48(v)


def _p11_body(b_ref, prev_ref, bst_ref, *rest):
    ws = [r[...] for r in rest[:-1]]
    y_ref = rest[-1]
    h = _in_gelu(prev_ref[...], b_ref[...], bst_ref[...])
    y_ref[...] = _mlp(h, ws)


def _merge_body(p_ref, *outs, want_sums, want_mean):
    p = p_ref[0] + p_ref[1]
    i = 0
    if want_sums:
        outs[i][...] = p
        i += 1
    if want_mean:
        outs[i][...] = p[:, :32] / jnp.maximum(p[:, 34:35], 1.0)


def _sc_scatter(vals, idx2d, zeros, cpt):
    mesh = plsc.VectorSubcoreMesh(core_axis_name="c", subcore_axis_name="s",
                                  num_cores=_NC, num_subcores=_NS)
    rpt = _TR // _NS

    @functools.partial(
        pl.kernel,
        out_type=jax.ShapeDtypeStruct((_NC, _TR, _VC), _F32),
        mesh=mesh,
        scratch_types=[
            pltpu.VMEM_SHARED((_TR, _VC), _F32),
            pltpu.VMEM((cpt, _CHUNK), jnp.int32),
            pltpu.VMEM((_CHUNK, _VC), _F32),
        ],
        compiler_params=pltpu.CompilerParams(use_tc_tiling_on_sc=False),
    )
    def scat(vals_hbm, idx_hbm, zeros_hbm, out_hbm, table, idx_v, buf):
        cc = lax.axis_index("c")
        ss = lax.axis_index("s")
        wid = ss * _NC + cc
        pltpu.sync_copy(zeros_hbm.at[pl.ds(ss * rpt, rpt)],
                        table.at[pl.ds(ss * rpt, rpt)])
        pltpu.sync_copy(idx_hbm.at[wid], idx_v)
        plsc.subcore_barrier()

        def step(j, cr):
            off = (wid * cpt + j) * _CHUNK
            pltpu.sync_copy(vals_hbm.at[pl.ds(off, _CHUNK)], buf)
            pltpu.sync_copy(buf, table.at[idx_v.at[j]], add=True)
            return cr

        lax.fori_loop(0, cpt, step, 0)
        plsc.subcore_barrier()
        pltpu.sync_copy(table.at[pl.ds(ss * rpt, rpt)],
                        out_hbm.at[cc, pl.ds(ss * rpt, rpt)])

    return scat(vals, idx2d, zeros)


def _sc_gather(tab, idx2d, npad, cpt):
    mesh = plsc.VectorSubcoreMesh(core_axis_name="c", subcore_axis_name="s",
                                  num_cores=_NC, num_subcores=_NS)

    @functools.partial(
        pl.kernel,
        out_type=jax.ShapeDtypeStruct((npad, 32), _F32),
        mesh=mesh,
        scratch_types=[
            pltpu.VMEM((cpt, _CHUNK), jnp.int32),
            pltpu.VMEM((_CHUNK, 32), _F32),
            pltpu.SemaphoreType.DMA,
        ],
        compiler_params=pltpu.CompilerParams(use_tc_tiling_on_sc=False),
    )
    def gat(tab_hbm, idx_hbm, out_hbm, idx_v, buf, sem):
        cc = lax.axis_index("c")
        ss = lax.axis_index("s")
        wid = ss * _NC + cc
        pltpu.sync_copy(idx_hbm.at[wid], idx_v)

        def step(j, cr):
            off = (wid * cpt + j) * _CHUNK
            pltpu.async_copy(tab_hbm.at[idx_v.at[j]], buf, sem).wait()
            pltpu.sync_copy(buf, out_hbm.at[pl.ds(off, _CHUNK)])
            return cr

        lax.fori_loop(0, cpt, step, 0)

    return gat(tab, idx2d)


def _mid_body(m1_ref, m3_ref, *rest):
    out_ref = rest[-1]
    loaded = iter([r[...] for r in rest[:-1]])

    def take4():
        return [next(loaded) for _ in range(4)]

    def ohm(ppos, pb, n):
        m = ppos.shape[0]
        cx = jnp.clip(jnp.floor(ppos[:, 0:1] * n).astype(jnp.int32), 0, n - 1)
        cy = jnp.clip(jnp.floor(ppos[:, 1:2] * n).astype(jnp.int32), 0, n - 1)
        sub = pb * (n * n) + cx * n + cy
        ids = lax.broadcasted_iota(jnp.int32, (m, _NB * n * n), 1)
        return (sub == ids).astype(_F32)

    def bhm(pb):
        ids = lax.broadcasted_iota(jnp.int32, (pb.shape[0], _NB), 1)
        return (pb == ids).astype(_F32)

    def dd(x, s_oh, t_oh, tpos, vws, tws):
        v = _mlp(x, vws)
        c = v.shape[1]
        ones = jnp.ones((x.shape[0], 1), _F32)
        sums = _dotg0(s_oh, jnp.concatenate([v, ones], axis=1))
        g = jnp.dot(t_oh, sums, preferred_element_type=_F32)
        mean = g[:, :c] / jnp.maximum(g[:, c:c + 1], 1.0)
        return _mlp(jnp.concatenate([tpos, mean], axis=1), tws)

    def inorm(x, bho):
        c = x.shape[1]
        ones = jnp.ones((x.shape[0], 1), _F32)
        s = _dotg0(bho, jnp.concatenate([x, x * x, ones], axis=1))
        row = jnp.dot(bho, s, preferred_element_type=_F32)
        cnt = jnp.maximum(row[:, 2 * c:2 * c + 1], 1.0)
        mean = row[:, :c] / cnt
        var = row[:, c:2 * c] / cnt - mean * mean
        return _gelu((x - mean) / jnp.sqrt(var + 1e-5))

    def blockf(x, spos, s_oh, s_bh, tpos, t_oh, t_bh):
        o = dd(x, s_oh, s_oh, spos, take4(), take4())
        o = inorm(o, s_bh)
        o = dd(o, s_oh, t_oh, tpos, take4(), take4())
        return inorm(o, t_bh)

    def pool(x, ppos, oh):
        ones = jnp.ones((x.shape[0], 1), _F32)
        ps = _dotg0(oh, jnp.concatenate([x, ppos, ones], axis=1))
        c = x.shape[1]
        cnt = jnp.maximum(ps[:, c + 2:c + 3], 1.0)
        return ps[:, :c] / cnt, ps[:, c:c + 2] / cnt

    m1 = m1_ref[...]
    m3 = m3_ref[...]
    cnt1 = jnp.maximum(m1[:_S, 34:35], 1.0)
    p1pos = m1[:_S, 32:34] / cnt1
    p1x = m3[:_S, 0:32] / cnt1
    p1b = lax.broadcasted_iota(jnp.int32, (1024, 1), 0) // 256
    p2b = lax.broadcasted_iota(jnp.int32, (256, 1), 0) // 64
    p3b = lax.broadcasted_iota(jnp.int32, (64, 1), 0) // 16
    p4b = lax.broadcasted_iota(jnp.int32, (16, 1), 0) // 4
    bh1, bh2, bh3, bh4 = bhm(p1b), bhm(p2b), bhm(p3b), bhm(p4b)

    oh_p1_8 = ohm(p1pos, p1b, 8)
    e2 = blockf(p1x, p1pos, oh_p1_8, bh1, p1pos, oh_p1_8, bh1)
    p2x, p2pos = pool(e2, p1pos, oh_p1_8)
    oh_p2_4 = ohm(p2pos, p2b, 4)
    e3 = blockf(p2x, p2pos, oh_p2_4, bh2, p2pos, oh_p2_4, bh2)
    p3x, p3pos = pool(e3, p2pos, oh_p2_4)
    oh_p3_2 = ohm(p3pos, p3b, 2)
    e4 = blockf(p3x, p3pos, oh_p3_2, bh3, p3pos, oh_p3_2, bh3)
    p4x, p4pos = pool(e4, p3pos, oh_p3_2)
    oh_p4_1 = ohm(p4pos, p4b, 1)
    bb = blockf(p4x, p4pos, oh_p4_1, bh4, p4pos, oh_p4_1, bh4)
    bb = blockf(bb, p4pos, oh_p4_1, bh4, p4pos, oh_p4_1, bh4)
    oh_p4_2 = ohm(p4pos, p4b, 2)
    d4 = blockf(jnp.concatenate([bb, p4x], axis=1), p4pos, oh_p4_2,
                bh4, p3pos, oh_p3_2, bh3)
    oh_p3_4 = ohm(p3pos, p3b, 4)
    d3 = blockf(jnp.concatenate([d4, p3x], axis=1), p3pos, oh_p3_4,
                bh3, p2pos, oh_p2_4, bh2)
    oh_p2_8 = ohm(p2pos, p2b, 8)
    d2 = blockf(jnp.concatenate([d3, p2x], axis=1), p2pos, oh_p2_8,
                bh2, p1pos, oh_p1_8, bh1)
    oh_p1_16 = ohm(p1pos, p1b, 16)
    o = dd(jnp.concatenate([d2, p1x], axis=1), oh_p1_16, oh_p1_16,
           p1pos, take4(), take4())
    h = inorm(o, bh1)
    v = _mlp(h, take4())
    ones = jnp.ones((1024, 1), _F32)
    sums = _dotg0(oh_p1_16, jnp.concatenate([v, ones], axis=1))
    out_ref[...] = sums[:, :32] / jnp.maximum(sums[:, 32:33], 1.0)


def _rows(c):
    return pl.BlockSpec((_R, c), lambda i: (i, 0))


def _full(a):
    nd = a.ndim
    return pl.BlockSpec(a.shape, lambda i: (0,) * nd)


def _sds(shape, dt=_F32):
    return jax.ShapeDtypeStruct(shape, dt)


def kernel(x, pos, batch, params):
    n = x.shape[0]
    npad = -(-n // _ALIGN) * _ALIGN
    nt = npad // _R
    cpt = npad // (_NW * _CHUNK)
    padn = npad - n
    xp = jnp.pad(x, ((0, padn), (0, 0)))
    posp = jnp.pad(pos, ((0, padn), (0, 0)))
    bp = jnp.pad(batch.astype(jnp.int32), (0, padn),
                 constant_values=_NB).reshape(npad, 1)
    zeros_tab = jnp.zeros((_TR, _VC), _F32)

    def call(body, ins, outs, out_specs):
        specs = []
        for a, kind in ins:
            specs.append(_rows(kind) if isinstance(kind, int) else _full(a))
        return pl.pallas_call(
            body,
            grid=(nt,),
            in_specs=specs,
            out_specs=out_specs,
            out_shape=outs,
        )(*[a for a, _ in ins])

    def merge(parts, want_sums, want_mean):
        outs, specs = [], []
        if want_sums:
            outs.append(_sds((_TR, _VC)))
            specs.append(_full(jnp.zeros((_TR, _VC))))
        if want_mean:
            outs.append(_sds((_TR, 32)))
            specs.append(_full(jnp.zeros((_TR, 32))))
        r = pl.pallas_call(
            functools.partial(_merge_body, want_sums=want_sums,
                              want_mean=want_mean),
            grid=(1,),
            in_specs=[_full(parts)],
            out_specs=specs,
            out_shape=outs,
        )(parts)
        return r if len(r) > 1 else r[0]

    p = params
    e1o1, e1o2 = p["enc1"]["op1"], p["enc1"]["op2"]
    d0o1, d0o2 = p["dec0"]["op1"], p["dec0"]["op2"]

    ws1 = (_mlpw(p["point_encode"]) + _mlpw(p["lift"]) + _mlpw(e1o1["op_v"]))
    x0, vals1, sub = call(
        _p1_body,
        [(xp, 128), (posp, 2), (bp, 1)] + [(w, None) for w in ws1],
        [_sds((npad, 96)), _sds((npad, _VC)), _sds((npad, 1), jnp.int32)],
        [_rows(96), _rows(_VC), _rows(1)],
    )
    idx2d = sub.reshape(_NW, cpt, _CHUNK)

    def gather_pass(gat, tws):
        return call(
            _g_body,
            [(posp, 2), (bp, 1), (gat, 32)] + [(w, None) for w in tws],
            [_sds((npad, 32)), _sds((8, 65))],
            [_rows(32), _full(jnp.zeros((8, 65)))],
        )

    def ns_pass(prev, bst, ws, with_x0=False):
        ins = [(bp, 1), (prev, 32), (bst, None)]
        if with_x0:
            ins.append((x0, 96))
        ins += [(w, None) for w in ws]
        return call(
            functools.partial(_ns_body, nws=len(ws), with_x0=with_x0),
            ins,
            _sds((npad, _VC)),
            _rows(_VC),
        )

    def seg_roundtrip(vals, want_sums=False):
        parts = _sc_scatter(vals, idx2d, zeros_tab, cpt)
        m = merge(parts, want_sums, True)
        sums, mean = m if want_sums else (None, m)
        return sums, _sc_gather(mean, idx2d, npad, cpt)

    merged1, g1 = seg_roundtrip(vals1, want_sums=True)
    out1, bst1 = gather_pass(g1, _mlpw(e1o1["op_tgt_kernel"]))
    vals2 = ns_pass(out1, bst1, _mlpw(e1o2["op_v"]))
    _, g2 = seg_roundtrip(vals2)
    out2, bst2 = gather_pass(g2, _mlpw(e1o2["op_tgt_kernel"]))
    vals3 = ns_pass(out2, bst2, [])
    parts3 = _sc_scatter(vals3, idx2d, zeros_tab, cpt)
    merged3 = merge(parts3, True, False)

    mid_ws = []
    for name in ["enc2", "enc3", "enc4", "bot1", "bot2",
                 "dec4", "dec3", "dec2"]:
        for opn in ["op1", "op2"]:
            mid_ws += _mlpw(p[name][opn]["op_v"])
            mid_ws += _mlpw(p[name][opn]["op_tgt_kernel"])
    mid_ws += _mlpw(p["dec1"]["op1"]["op_v"])
    mid_ws += _mlpw(p["dec1"]["op1"]["op_tgt_kernel"])
    mid_ws += _mlpw(p["dec1"]["op2"]["op_v"])
    d1tab = call(
        _mid_body,
        [(merged1, None), (merged3, None)] + [(w, None) for w in mid_ws],
        _sds((_S, 32)),
        _full(jnp.zeros((_S, 32))),
    )
    d1pad = jnp.pad(d1tab, ((0, _TR - _S), (0, 0)))
    gmid = _sc_gather(d1pad, idx2d, npad, cpt)

    out_d1, bst3 = gather_pass(gmid, _mlpw(p["dec1"]["op2"]["op_tgt_kernel"]))
    vals4 = ns_pass(out_d1, bst3, _mlpw(d0o1["op_v"]), with_x0=True)
    _, g4 = seg_roundtrip(vals4)
    out_e, bst4 = gather_pass(g4, _mlpw(d0o1["op_tgt_kernel"]))
    vals5 = ns_pass(out_e, bst4, _mlpw(d0o2["op_v"]))
    _, g5 = seg_roundtrip(vals5)
    out_f, bst5 = gather_pass(g5, _mlpw(d0o2["op_tgt_kernel"]))

    y = call(
        _p11_body,
        [(bp, 1), (out_f, 32), (bst5, None)] + [(w, None)
                                                for w in _mlpw(p["project"])],
        _sds((npad, 128)),
        _rows(128),
    )
    return y[:n]


# speedup vs baseline: 1.8647x; 1.0693x over previous
import functools

import jax
import jax.numpy as jnp
from jax import lax
from jax.experimental import pallas as pl
from jax.experimental.pallas import tpu as pltpu
from jax.experimental.pallas import tpu_sc as plsc

_NB = 4
_G = 16
_S = _NB * _G * _G
_R = 7168
_F32 = jnp.float32

_NC, _NS = 2, 16
_NW = _NC * _NS
_CHUNK = 112
_ALIGN = _R
_TR = 1152
_VC = 48


def _gelu(x):
    return x * 0.5 * (1.0 + lax.erf(x * 0.7071067811865476))


def _mlpw(p):
    ws = []
    for layer in p:
        ws.append(layer["W"])
        ws.append(layer["b"].reshape(1, -1))
    return ws


def _mlp(x, ws):
    n = len(ws) // 2
    for i in range(n):
        x = jnp.dot(x, ws[2 * i], preferred_element_type=_F32) + ws[2 * i + 1]
        if i < n - 1:
            x = _gelu(x)
    return x


def _dotg0(a, b):
    return lax.dot_general(a, b, (((0,), (0,)), ((), ())),
                           preferred_element_type=_F32)


def _subid(pos, batch):
    cx = jnp.clip(jnp.floor(pos[:, 0:1] * _G).astype(jnp.int32), 0, _G - 1)
    cy = jnp.clip(jnp.floor(pos[:, 1:2] * _G).astype(jnp.int32), 0, _G - 1)
    return batch * (_G * _G) + cx * _G + cy


def _bh(batch):
    ids = lax.broadcasted_iota(jnp.int32, (batch.shape[0], 8), 1)
    return (batch == ids).astype(_F32)


def _in_gelu(x, batch, bstats):
    C = x.shape[1]
    s = jnp.dot(_bh(batch), bstats, preferred_element_type=_F32)
    cnt = jnp.maximum(s[:, 2 * C:2 * C + 1], 1.0)
    mean = s[:, :C] / cnt
    var = s[:, C:2 * C] / cnt - mean * mean
    return _gelu((x - mean) / jnp.sqrt(var + 1e-5))


def _acc_init(ref):
    @pl.when(pl.program_id(0) == 0)
    def _():
        ref[...] = jnp.zeros_like(ref)


def _vals48(v, pos=None):
    r = v.shape[0]
    ones = jnp.ones((r, 1), _F32)
    mid = pos if pos is not None else jnp.zeros((r, 2), _F32)
    return jnp.concatenate([v, mid, ones, jnp.zeros((r, _VC - 35), _F32)],
                           axis=1)


def _p1_body(x_ref, pos_ref, b_ref, *rest):
    ws = [r[...] for r in rest[:-3]]
    x0_ref, vals_ref, sub_ref = rest[-3], rest[-2], rest[-1]
    pos, b = pos_ref[...], b_ref[...]
    pe = _mlp(pos, ws[0:6])
    lf = _mlp(x_ref[...], ws[6:10])
    x0 = jnp.concatenate([pe, lf], axis=1)
    x0_ref[...] = x0
    v = _mlp(x0, ws[10:14])
    vals_ref[...] = _vals48(v, pos)
    sub_ref[...] = _subid(pos, b)


def _g_body(pos_ref, b_ref, g_ref, *rest):
    ws = [r[...] for r in rest[:-2]]
    out_ref, bst_ref = rest[-2], rest[-1]
    pos, b = pos_ref[...], b_ref[...]
    out = _mlp(jnp.concatenate([pos, g_ref[...]], axis=1), ws)
    out_ref[...] = out
    ones = jnp.ones((pos.shape[0], 1), _F32)
    sb = jnp.concatenate([out, out * out, ones], axis=1)
    _acc_init(bst_ref)
    bst_ref[...] += _dotg0(_bh(b), sb)


def _ns_body(b_ref, prev_ref, bst_ref, *rest, nws, with_x0):
    k = 1 if with_x0 else 0
    x0 = rest[0][...] if with_x0 else None
    ws = [r[...] for r in rest[k:k + nws]]
    vals_ref = rest[-1]
    b = b_ref[...]
    h = _in_gelu(prev_ref[...], b, bst_ref[...])
    if with_x0:
        h = jnp.concatenate([h, x0], axis=1)
    v = _mlp(h, ws) if nws else h
    vals_ref[...] = _vals48(v)


def _p11_body(b_ref, prev_ref, bst_ref, *rest):
    ws = [r[...] for r in rest[:-1]]
    y_ref = rest[-1]
    h = _in_gelu(prev_ref[...], b_ref[...], bst_ref[...])
    y_ref[...] = _mlp(h, ws)


def _merge_body(p_ref, *outs, want_sums, want_mean):
    p = p_ref[0] + p_ref[1]
    i = 0
    if want_sums:
        outs[i][...] = p
        i += 1
    if want_mean:
        outs[i][...] = p[:, :32] / jnp.maximum(p[:, 34:35], 1.0)


def _sc_scatter(vals, idx2d, zeros, cpt):
    mesh = plsc.VectorSubcoreMesh(core_axis_name="c", subcore_axis_name="s",
                                  num_cores=_NC, num_subcores=_NS)
    rpt = _TR // _NS

    half = cpt // 2
    rows_h = half * _CHUNK

    @functools.partial(
        pl.kernel,
        out_type=jax.ShapeDtypeStruct((_NC, _TR, _VC), _F32),
        mesh=mesh,
        scratch_types=[
            pltpu.VMEM_SHARED((_TR, _VC), _F32),
            pltpu.VMEM((cpt, _CHUNK), jnp.int32),
            pltpu.VMEM((rows_h, _VC), _F32),
            pltpu.SemaphoreType.DMA,
        ],
        compiler_params=pltpu.CompilerParams(use_tc_tiling_on_sc=False),
    )
    def scat(vals_hbm, idx_hbm, zeros_hbm, out_hbm, table, idx_v, buf, sem):
        cc = lax.axis_index("c")
        ss = lax.axis_index("s")
        wid = ss * _NC + cc
        pltpu.sync_copy(zeros_hbm.at[pl.ds(ss * rpt, rpt)],
                        table.at[pl.ds(ss * rpt, rpt)])
        pltpu.sync_copy(idx_hbm.at[wid], idx_v)
        plsc.subcore_barrier()
        base = wid * cpt * _CHUNK

        def phase(ph, cr):
            src = vals_hbm.at[pl.ds(base + ph * rows_h, rows_h)]
            pltpu.sync_copy(src, buf)

            def fire(j, c2):
                pltpu.async_copy(buf.at[pl.ds(j * _CHUNK, _CHUNK)],
                                 table.at[idx_v.at[ph * half + j]],
                                 sem, add=True)
                return c2

            lax.fori_loop(0, half, fire, 0)
            pltpu.make_async_copy(src, buf, sem).wait()
            return cr

        lax.fori_loop(0, 2, phase, 0)
        plsc.subcore_barrier()
        pltpu.sync_copy(table.at[pl.ds(ss * rpt, rpt)],
                        out_hbm.at[cc, pl.ds(ss * rpt, rpt)])

    return scat(vals, idx2d, zeros)


def _sc_gather(tab, idx2d, npad, cpt):
    mesh = plsc.VectorSubcoreMesh(core_axis_name="c", subcore_axis_name="s",
                                  num_cores=_NC, num_subcores=_NS)

    rows_t = cpt * _CHUNK

    @functools.partial(
        pl.kernel,
        out_type=jax.ShapeDtypeStruct((npad, 32), _F32),
        mesh=mesh,
        scratch_types=[
            pltpu.VMEM((cpt, _CHUNK), jnp.int32),
            pltpu.VMEM((rows_t, 32), _F32),
            pltpu.SemaphoreType.DMA,
        ],
        compiler_params=pltpu.CompilerParams(use_tc_tiling_on_sc=False),
    )
    def gat(tab_hbm, idx_hbm, out_hbm, idx_v, buf, sem):
        cc = lax.axis_index("c")
        ss = lax.axis_index("s")
        wid = ss * _NC + cc
        base = wid * rows_t
        pltpu.sync_copy(idx_hbm.at[wid], idx_v)

        def fire(j, cr):
            pltpu.async_copy(tab_hbm.at[idx_v.at[j]],
                             buf.at[pl.ds(j * _CHUNK, _CHUNK)], sem)
            return cr

        lax.fori_loop(0, cpt, fire, 0)
        pltpu.make_async_copy(out_hbm.at[pl.ds(base, rows_t)], buf, sem).wait()
        pltpu.sync_copy(buf, out_hbm.at[pl.ds(base, rows_t)])

    return gat(tab, idx2d)


def _mid_body(m1_ref, m3_ref, *rest):
    out_ref = rest[-1]
    loaded = iter([r[...] for r in rest[:-1]])

    def take4():
        return [next(loaded) for _ in range(4)]

    def ohm(ppos, pb, n):
        m = ppos.shape[0]
        cx = jnp.clip(jnp.floor(ppos[:, 0:1] * n).astype(jnp.int32), 0, n - 1)
        cy = jnp.clip(jnp.floor(ppos[:, 1:2] * n).astype(jnp.int32), 0, n - 1)
        sub = pb * (n * n) + cx * n + cy
        ids = lax.broadcasted_iota(jnp.int32, (m, _NB * n * n), 1)
        return (sub == ids).astype(_F32)

    def bhm(pb):
        ids = lax.broadcasted_iota(jnp.int32, (pb.shape[0], _NB), 1)
        return (pb == ids).astype(_F32)

    def dd(x, s_oh, t_oh, tpos, vws, tws):
        v = _mlp(x, vws)
        c = v.shape[1]
        ones = jnp.ones((x.shape[0], 1), _F32)
        sums = _dotg0(s_oh, jnp.concatenate([v, ones], axis=1))
        g = jnp.dot(t_oh, sums, preferred_element_type=_F32)
        mean = g[:, :c] / jnp.maximum(g[:, c:c + 1], 1.0)
        return _mlp(jnp.concatenate([tpos, mean], axis=1), tws)

    def inorm(x, bho):
        c = x.shape[1]
        ones = jnp.ones((x.shape[0], 1), _F32)
        s = _dotg0(bho, jnp.concatenate([x, x * x, ones], axis=1))
        row = jnp.dot(bho, s, preferred_element_type=_F32)
        cnt = jnp.maximum(row[:, 2 * c:2 * c + 1], 1.0)
        mean = row[:, :c] / cnt
        var = row[:, c:2 * c] / cnt - mean * mean
        return _gelu((x - mean) / jnp.sqrt(var + 1e-5))

    def blockf(x, spos, s_oh, s_bh, tpos, t_oh, t_bh):
        o = dd(x, s_oh, s_oh, spos, take4(), take4())
        o = inorm(o, s_bh)
        o = dd(o, s_oh, t_oh, tpos, take4(), take4())
        return inorm(o, t_bh)

    def pool(x, ppos, oh):
        ones = jnp.ones((x.shape[0], 1), _F32)
        ps = _dotg0(oh, jnp.concatenate([x, ppos, ones], axis=1))
        c = x.shape[1]
        cnt = jnp.maximum(ps[:, c + 2:c + 3], 1.0)
        return ps[:, :c] / cnt, ps[:, c:c + 2] / cnt

    m1 = m1_ref[...]
    m3 = m3_ref[...]
    cnt1 = jnp.maximum(m1[:_S, 34:35], 1.0)
    p1pos = m1[:_S, 32:34] / cnt1
    p1x = m3[:_S, 0:32] / cnt1
    p1b = lax.broadcasted_iota(jnp.int32, (1024, 1), 0) // 256
    p2b = lax.broadcasted_iota(jnp.int32, (256, 1), 0) // 64
    p3b = lax.broadcasted_iota(jnp.int32, (64, 1), 0) // 16
    p4b = lax.broadcasted_iota(jnp.int32, (16, 1), 0) // 4
    bh1, bh2, bh3, bh4 = bhm(p1b), bhm(p2b), bhm(p3b), bhm(p4b)

    oh_p1_8 = ohm(p1pos, p1b, 8)
    e2 = blockf(p1x, p1pos, oh_p1_8, bh1, p1pos, oh_p1_8, bh1)
    p2x, p2pos = pool(e2, p1pos, oh_p1_8)
    oh_p2_4 = ohm(p2pos, p2b, 4)
    e3 = blockf(p2x, p2pos, oh_p2_4, bh2, p2pos, oh_p2_4, bh2)
    p3x, p3pos = pool(e3, p2pos, oh_p2_4)
    oh_p3_2 = ohm(p3pos, p3b, 2)
    e4 = blockf(p3x, p3pos, oh_p3_2, bh3, p3pos, oh_p3_2, bh3)
    p4x, p4pos = pool(e4, p3pos, oh_p3_2)
    oh_p4_1 = ohm(p4pos, p4b, 1)
    bb = blockf(p4x, p4pos, oh_p4_1, bh4, p4pos, oh_p4_1, bh4)
    bb = blockf(bb, p4pos, oh_p4_1, bh4, p4pos, oh_p4_1, bh4)
    oh_p4_2 = ohm(p4pos, p4b, 2)
    d4 = blockf(jnp.concatenate([bb, p4x], axis=1), p4pos, oh_p4_2,
                bh4, p3pos, oh_p3_2, bh3)
    oh_p3_4 = ohm(p3pos, p3b, 4)
    d3 = blockf(jnp.concatenate([d4, p3x], axis=1), p3pos, oh_p3_4,
                bh3, p2pos, oh_p2_4, bh2)
    oh_p2_8 = ohm(p2pos, p2b, 8)
    d2 = blockf(jnp.concatenate([d3, p2x], axis=1), p2pos, oh_p2_8,
                bh2, p1pos, oh_p1_8, bh1)
    oh_p1_16 = ohm(p1pos, p1b, 16)
    o = dd(jnp.concatenate([d2, p1x], axis=1), oh_p1_16, oh_p1_16,
           p1pos, take4(), take4())
    h = inorm(o, bh1)
    v = _mlp(h, take4())
    ones = jnp.ones((1024, 1), _F32)
    sums = _dotg0(oh_p1_16, jnp.concatenate([v, ones], axis=1))
    out_ref[...] = sums[:, :32] / jnp.maximum(sums[:, 32:33], 1.0)


def _rows(c):
    return pl.BlockSpec((_R, c), lambda i: (i, 0))


def _full(a):
    nd = a.ndim
    return pl.BlockSpec(a.shape, lambda i: (0,) * nd)


def _sds(shape, dt=_F32):
    return jax.ShapeDtypeStruct(shape, dt)


def kernel(x, pos, batch, params):
    n = x.shape[0]
    npad = -(-n // _ALIGN) * _ALIGN
    nt = npad // _R
    cpt = npad // (_NW * _CHUNK)
    padn = npad - n
    xp = jnp.pad(x, ((0, padn), (0, 0)))
    posp = jnp.pad(pos, ((0, padn), (0, 0)))
    bp = jnp.pad(batch.astype(jnp.int32), (0, padn),
                 constant_values=_NB).reshape(npad, 1)
    zeros_tab = jnp.zeros((_TR, _VC), _F32)

    def call(body, ins, outs, out_specs):
        specs = []
        for a, kind in ins:
            specs.append(_rows(kind) if isinstance(kind, int) else _full(a))
        return pl.pallas_call(
            body,
            grid=(nt,),
            in_specs=specs,
            out_specs=out_specs,
            out_shape=outs,
        )(*[a for a, _ in ins])

    def merge(parts, want_sums, want_mean):
        outs, specs = [], []
        if want_sums:
            outs.append(_sds((_TR, _VC)))
            specs.append(_full(jnp.zeros((_TR, _VC))))
        if want_mean:
            outs.append(_sds((_TR, 32)))
            specs.append(_full(jnp.zeros((_TR, 32))))
        r = pl.pallas_call(
            functools.partial(_merge_body, want_sums=want_sums,
                              want_mean=want_mean),
            grid=(1,),
            in_specs=[_full(parts)],
            out_specs=specs,
            out_shape=outs,
        )(parts)
        return r if len(r) > 1 else r[0]

    p = params
    e1o1, e1o2 = p["enc1"]["op1"], p["enc1"]["op2"]
    d0o1, d0o2 = p["dec0"]["op1"], p["dec0"]["op2"]

    ws1 = (_mlpw(p["point_encode"]) + _mlpw(p["lift"]) + _mlpw(e1o1["op_v"]))
    x0, vals1, sub = call(
        _p1_body,
        [(xp, 128), (posp, 2), (bp, 1)] + [(w, None) for w in ws1],
        [_sds((npad, 96)), _sds((npad, _VC)), _sds((npad, 1), jnp.int32)],
        [_rows(96), _rows(_VC), _rows(1)],
    )
    idx2d = sub.reshape(_NW, cpt, _CHUNK)

    def gather_pass(gat, tws):
        return call(
            _g_body,
            [(posp, 2), (bp, 1), (gat, 32)] + [(w, None) for w in tws],
            [_sds((npad, 32)), _sds((8, 65))],
            [_rows(32), _full(jnp.zeros((8, 65)))],
        )

    def ns_pass(prev, bst, ws, with_x0=False):
        ins = [(bp, 1), (prev, 32), (bst, None)]
        if with_x0:
            ins.append((x0, 96))
        ins += [(w, None) for w in ws]
        return call(
            functools.partial(_ns_body, nws=len(ws), with_x0=with_x0),
            ins,
            _sds((npad, _VC)),
            _rows(_VC),
        )

    def seg_roundtrip(vals, want_sums=False):
        parts = _sc_scatter(vals, idx2d, zeros_tab, cpt)
        m = merge(parts, want_sums, True)
        sums, mean = m if want_sums else (None, m)
        return sums, _sc_gather(mean, idx2d, npad, cpt)

    merged1, g1 = seg_roundtrip(vals1, want_sums=True)
    out1, bst1 = gather_pass(g1, _mlpw(e1o1["op_tgt_kernel"]))
    vals2 = ns_pass(out1, bst1, _mlpw(e1o2["op_v"]))
    _, g2 = seg_roundtrip(vals2)
    out2, bst2 = gather_pass(g2, _mlpw(e1o2["op_tgt_kernel"]))
    vals3 = ns_pass(out2, bst2, [])
    parts3 = _sc_scatter(vals3, idx2d, zeros_tab, cpt)
    merged3 = merge(parts3, True, False)

    mid_ws = []
    for name in ["enc2", "enc3", "enc4", "bot1", "bot2",
                 "dec4", "dec3", "dec2"]:
        for opn in ["op1", "op2"]:
            mid_ws += _mlpw(p[name][opn]["op_v"])
            mid_ws += _mlpw(p[name][opn]["op_tgt_kernel"])
    mid_ws += _mlpw(p["dec1"]["op1"]["op_v"])
    mid_ws += _mlpw(p["dec1"]["op1"]["op_tgt_kernel"])
    mid_ws += _mlpw(p["dec1"]["op2"]["op_v"])
    d1tab = call(
        _mid_body,
        [(merged1, None), (merged3, None)] + [(w, None) for w in mid_ws],
        _sds((_S, 32)),
        _full(jnp.zeros((_S, 32))),
    )
    d1pad = jnp.pad(d1tab, ((0, _TR - _S), (0, 0)))
    gmid = _sc_gather(d1pad, idx2d, npad, cpt)

    out_d1, bst3 = gather_pass(gmid, _mlpw(p["dec1"]["op2"]["op_tgt_kernel"]))
    vals4 = ns_pass(out_d1, bst3, _mlpw(d0o1["op_v"]), with_x0=True)
    _, g4 = seg_roundtrip(vals4)
    out_e, bst4 = gather_pass(g4, _mlpw(d0o1["op_tgt_kernel"]))
    vals5 = ns_pass(out_e, bst4, _mlpw(d0o2["op_v"]))
    _, g5 = seg_roundtrip(vals5)
    out_f, bst5 = gather_pass(g5, _mlpw(d0o2["op_tgt_kernel"]))

    y = call(
        _p11_body,
        [(bp, 1), (out_f, 32), (bst5, None)] + [(w, None)
                                                for w in _mlpw(p["project"])],
        _sds((npad, 128)),
        _rows(128),
    )
    return y[:n]


# speedup vs baseline: 2.2668x; 1.2156x over previous
import functools

import jax
import jax.numpy as jnp
from jax import lax
from jax.experimental import pallas as pl
from jax.experimental.pallas import tpu as pltpu
from jax.experimental.pallas import tpu_sc as plsc

_NB = 4
_G = 16
_S = _NB * _G * _G
_R = 7168
_F32 = jnp.float32

_NC, _NS = 2, 16
_NW = _NC * _NS
_CHUNK = 112
_ALIGN = _R
_TR = 1152
_VC = 48


def _gelu(x):
    return x * 0.5 * (1.0 + lax.erf(x * 0.7071067811865476))


def _mlpw(p):
    ws = []
    for layer in p:
        ws.append(layer["W"])
        ws.append(layer["b"].reshape(1, -1))
    return ws


def _mlp(x, ws):
    n = len(ws) // 2
    for i in range(n):
        x = jnp.dot(x, ws[2 * i], preferred_element_type=_F32) + ws[2 * i + 1]
        if i < n - 1:
            x = _gelu(x)
    return x


def _dotg0(a, b):
    return lax.dot_general(a, b, (((0,), (0,)), ((), ())),
                           preferred_element_type=_F32)


def _subid(pos, batch):
    cx = jnp.clip(jnp.floor(pos[:, 0:1] * _G).astype(jnp.int32), 0, _G - 1)
    cy = jnp.clip(jnp.floor(pos[:, 1:2] * _G).astype(jnp.int32), 0, _G - 1)
    return batch * (_G * _G) + cx * _G + cy


def _bh(batch):
    ids = lax.broadcasted_iota(jnp.int32, (batch.shape[0], 8), 1)
    return (batch == ids).astype(_F32)


def _in_gelu(x, batch, bstats):
    C = x.shape[1]
    s = jnp.dot(_bh(batch), bstats, preferred_element_type=_F32)
    cnt = jnp.maximum(s[:, 2 * C:2 * C + 1], 1.0)
    mean = s[:, :C] / cnt
    var = s[:, C:2 * C] / cnt - mean * mean
    return _gelu((x - mean) / jnp.sqrt(var + 1e-5))


def _acc_init(ref):
    @pl.when(pl.program_id(0) == 0)
    def _():
        ref[...] = jnp.zeros_like(ref)


def _vals48(v, pos=None):
    r = v.shape[0]
    ones = jnp.ones((r, 1), _F32)
    mid = pos if pos is not None else jnp.zeros((r, 2), _F32)
    return jnp.concatenate([v, mid, ones, jnp.zeros((r, _VC - 35), _F32)],
                           axis=1)


def _p1_body(x_ref, pos_ref, b_ref, *rest):
    ws = [r[...] for r in rest[:-3]]
    x0_ref, vals_ref, sub_ref = rest[-3], rest[-2], rest[-1]
    pos, b = pos_ref[...], b_ref[...]
    pe = _mlp(pos, ws[0:6])
    lf = _mlp(x_ref[...], ws[6:10])
    x0 = jnp.concatenate([pe, lf], axis=1)
    x0_ref[...] = x0
    v = _mlp(x0, ws[10:14])
    vals_ref[...] = _vals48(v, pos)
    sub_ref[...] = _subid(pos, b)


def _g_body(pos_ref, b_ref, g_ref, *rest):
    ws = [r[...] for r in rest[:-2]]
    out_ref, bst_ref = rest[-2], rest[-1]
    pos, b = pos_ref[...], b_ref[...]
    out = _mlp(jnp.concatenate([pos, g_ref[...]], axis=1), ws)
    out_ref[...] = out
    ones = jnp.ones((pos.shape[0], 1), _F32)
    sb = jnp.concatenate([out, out * out, ones], axis=1)
    _acc_init(bst_ref)
    bst_ref[...] += _dotg0(_bh(b), sb)


def _ns_body(b_ref, prev_ref, bst_ref, *rest, nws, with_x0):
    k = 1 if with_x0 else 0
    x0 = rest[0][...] if with_x0 else None
    ws = [r[...] for r in rest[k:k + nws]]
    vals_ref = rest[-1]
    b = b_ref[...]
    h = _in_gelu(prev_ref[...], b, bst_ref[...])
    if with_x0:
        h = jnp.concatenate([h, x0], axis=1)
    v = _mlp(h, ws) if nws else h
    vals_ref[...] = _vals48(v)


def _p11_body(b_ref, prev_ref, bst_ref, *rest):
    ws = [r[...] for r in rest[:-1]]
    y_ref = rest[-1]
    h = _in_gelu(prev_ref[...], b_ref[...], bst_ref[...])
    y_ref[...] = _mlp(h, ws)


def _merge_body(p_ref, *outs, want_sums, want_mean):
    p = p_ref[0] + p_ref[1]
    i = 0
    if want_sums:
        outs[i][...] = p
        i += 1
    if want_mean:
        outs[i][...] = p[:, :32] / jnp.maximum(p[:, 34:35], 1.0)


def _sc_scatter(vals, idx2d, zeros, cpt):
    mesh = plsc.VectorSubcoreMesh(core_axis_name="c", subcore_axis_name="s",
                                  num_cores=_NC, num_subcores=_NS)
    rpt = _TR // _NS

    half = cpt // 2
    rows_h = half * _CHUNK

    @functools.partial(
        pl.kernel,
        out_type=jax.ShapeDtypeStruct((_NC, _TR, _VC), _F32),
        mesh=mesh,
        scratch_types=[
            pltpu.VMEM_SHARED((_TR, _VC), _F32),
            pltpu.VMEM((cpt, _CHUNK), jnp.int32),
            pltpu.VMEM((rows_h, _VC), _F32),
            pltpu.SemaphoreType.DMA,
        ],
        compiler_params=pltpu.CompilerParams(use_tc_tiling_on_sc=False),
    )
    def scat(vals_hbm, idx_hbm, zeros_hbm, out_hbm, table, idx_v, buf, sem):
        cc = lax.axis_index("c")
        ss = lax.axis_index("s")
        wid = ss * _NC + cc
        pltpu.sync_copy(zeros_hbm.at[pl.ds(ss * rpt, rpt)],
                        table.at[pl.ds(ss * rpt, rpt)])
        pltpu.sync_copy(idx_hbm.at[wid], idx_v)
        plsc.subcore_barrier()
        base = wid * cpt * _CHUNK

        def phase(ph, cr):
            src = vals_hbm.at[pl.ds(base + ph * rows_h, rows_h)]
            pltpu.sync_copy(src, buf)

            def fire(j, c2):
                pltpu.async_copy(buf.at[pl.ds(j * _CHUNK, _CHUNK)],
                                 table.at[idx_v.at[ph * half + j]],
                                 sem, add=True)
                return c2

            lax.fori_loop(0, half, fire, 0)
            pltpu.make_async_copy(src, buf, sem).wait()
            return cr

        lax.fori_loop(0, 2, phase, 0)
        plsc.subcore_barrier()
        pltpu.sync_copy(table.at[pl.ds(ss * rpt, rpt)],
                        out_hbm.at[cc, pl.ds(ss * rpt, rpt)])

    return scat(vals, idx2d, zeros)


def _sc_gather(tab, idx2d, npad, cpt):
    mesh = plsc.VectorSubcoreMesh(core_axis_name="c", subcore_axis_name="s",
                                  num_cores=_NC, num_subcores=_NS)

    rows_t = cpt * _CHUNK

    @functools.partial(
        pl.kernel,
        out_type=jax.ShapeDtypeStruct((npad, 32), _F32),
        mesh=mesh,
        scratch_types=[
            pltpu.VMEM((cpt, _CHUNK), jnp.int32),
            pltpu.VMEM((rows_t, 32), _F32),
            pltpu.SemaphoreType.DMA,
        ],
        compiler_params=pltpu.CompilerParams(use_tc_tiling_on_sc=False),
    )
    def gat(tab_hbm, idx_hbm, out_hbm, idx_v, buf, sem):
        cc = lax.axis_index("c")
        ss = lax.axis_index("s")
        wid = ss * _NC + cc
        base = wid * rows_t
        pltpu.sync_copy(idx_hbm.at[wid], idx_v)

        def fire(j, cr):
            pltpu.async_copy(tab_hbm.at[idx_v.at[j]],
                             buf.at[pl.ds(j * _CHUNK, _CHUNK)], sem)
            return cr

        lax.fori_loop(0, cpt, fire, 0)
        pltpu.make_async_copy(out_hbm.at[pl.ds(base, rows_t)], buf, sem).wait()
        pltpu.sync_copy(buf, out_hbm.at[pl.ds(base, rows_t)])

    return gat(tab, idx2d)


def _mid_body(m1_ref, p3_ref, *rest):
    out_ref = rest[-1]
    loaded = iter([r[...] for r in rest[:-1]])

    def take4():
        return [next(loaded) for _ in range(4)]

    def ohm(ppos, pb, n):
        m = ppos.shape[0]
        cx = jnp.clip(jnp.floor(ppos[:, 0:1] * n).astype(jnp.int32), 0, n - 1)
        cy = jnp.clip(jnp.floor(ppos[:, 1:2] * n).astype(jnp.int32), 0, n - 1)
        sub = pb * (n * n) + cx * n + cy
        ids = lax.broadcasted_iota(jnp.int32, (m, _NB * n * n), 1)
        return (sub == ids).astype(_F32)

    def bhm(pb):
        ids = lax.broadcasted_iota(jnp.int32, (pb.shape[0], _NB), 1)
        return (pb == ids).astype(_F32)

    def dd(x, s_oh, t_oh, tpos, vws, tws):
        v = _mlp(x, vws)
        c = v.shape[1]
        ones = jnp.ones((x.shape[0], 1), _F32)
        sums = _dotg0(s_oh, jnp.concatenate([v, ones], axis=1))
        g = jnp.dot(t_oh, sums, preferred_element_type=_F32)
        mean = g[:, :c] / jnp.maximum(g[:, c:c + 1], 1.0)
        return _mlp(jnp.concatenate([tpos, mean], axis=1), tws)

    def inorm(x, bho):
        c = x.shape[1]
        ones = jnp.ones((x.shape[0], 1), _F32)
        s = _dotg0(bho, jnp.concatenate([x, x * x, ones], axis=1))
        row = jnp.dot(bho, s, preferred_element_type=_F32)
        cnt = jnp.maximum(row[:, 2 * c:2 * c + 1], 1.0)
        mean = row[:, :c] / cnt
        var = row[:, c:2 * c] / cnt - mean * mean
        return _gelu((x - mean) / jnp.sqrt(var + 1e-5))

    def blockf(x, spos, s_oh, s_bh, tpos, t_oh, t_bh):
        o = dd(x, s_oh, s_oh, spos, take4(), take4())
        o = inorm(o, s_bh)
        o = dd(o, s_oh, t_oh, tpos, take4(), take4())
        return inorm(o, t_bh)

    def pool(x, ppos, oh):
        ones = jnp.ones((x.shape[0], 1), _F32)
        ps = _dotg0(oh, jnp.concatenate([x, ppos, ones], axis=1))
        c = x.shape[1]
        cnt = jnp.maximum(ps[:, c + 2:c + 3], 1.0)
        return ps[:, :c] / cnt, ps[:, c:c + 2] / cnt

    m1 = m1_ref[...]
    m3 = p3_ref[0] + p3_ref[1]
    cnt1 = jnp.maximum(m1[:_S, 34:35], 1.0)
    p1pos = m1[:_S, 32:34] / cnt1
    p1x = m3[:_S, 0:32] / cnt1
    p1b = lax.broadcasted_iota(jnp.int32, (1024, 1), 0) // 256
    p2b = lax.broadcasted_iota(jnp.int32, (256, 1), 0) // 64
    p3b = lax.broadcasted_iota(jnp.int32, (64, 1), 0) // 16
    p4b = lax.broadcasted_iota(jnp.int32, (16, 1), 0) // 4
    bh1, bh2, bh3, bh4 = bhm(p1b), bhm(p2b), bhm(p3b), bhm(p4b)

    oh_p1_8 = ohm(p1pos, p1b, 8)
    e2 = blockf(p1x, p1pos, oh_p1_8, bh1, p1pos, oh_p1_8, bh1)
    p2x, p2pos = pool(e2, p1pos, oh_p1_8)
    oh_p2_4 = ohm(p2pos, p2b, 4)
    e3 = blockf(p2x, p2pos, oh_p2_4, bh2, p2pos, oh_p2_4, bh2)
    p3x, p3pos = pool(e3, p2pos, oh_p2_4)
    oh_p3_2 = ohm(p3pos, p3b, 2)
    e4 = blockf(p3x, p3pos, oh_p3_2, bh3, p3pos, oh_p3_2, bh3)
    p4x, p4pos = pool(e4, p3pos, oh_p3_2)
    oh_p4_1 = ohm(p4pos, p4b, 1)
    bb = blockf(p4x, p4pos, oh_p4_1, bh4, p4pos, oh_p4_1, bh4)
    bb = blockf(bb, p4pos, oh_p4_1, bh4, p4pos, oh_p4_1, bh4)
    oh_p4_2 = ohm(p4pos, p4b, 2)
    d4 = blockf(jnp.concatenate([bb, p4x], axis=1), p4pos, oh_p4_2,
                bh4, p3pos, oh_p3_2, bh3)
    oh_p3_4 = ohm(p3pos, p3b, 4)
    d3 = blockf(jnp.concatenate([d4, p3x], axis=1), p3pos, oh_p3_4,
                bh3, p2pos, oh_p2_4, bh2)
    oh_p2_8 = ohm(p2pos, p2b, 8)
    d2 = blockf(jnp.concatenate([d3, p2x], axis=1), p2pos, oh_p2_8,
                bh2, p1pos, oh_p1_8, bh1)
    oh_p1_16 = ohm(p1pos, p1b, 16)
    o = dd(jnp.concatenate([d2, p1x], axis=1), oh_p1_16, oh_p1_16,
           p1pos, take4(), take4())
    h = inorm(o, bh1)
    v = _mlp(h, take4())
    ones = jnp.ones((1024, 1), _F32)
    sums = _dotg0(oh_p1_16, jnp.concatenate([v, ones], axis=1))
    out_ref[...] = sums[:, :32] / jnp.maximum(sums[:, 32:33], 1.0)


def _rows(c, r=_R):
    return pl.BlockSpec((r, c), lambda i: (i, 0))


def _full(a):
    nd = a.ndim
    return pl.BlockSpec(a.shape, lambda i: (0,) * nd)


def _sds(shape, dt=_F32):
    return jax.ShapeDtypeStruct(shape, dt)


def kernel(x, pos, batch, params):
    n = x.shape[0]
    npad = -(-n // _ALIGN) * _ALIGN
    nt = npad // _R
    cpt = npad // (_NW * _CHUNK)
    padn = npad - n
    xp = jnp.pad(x, ((0, padn), (0, 0)))
    posp = jnp.pad(pos, ((0, padn), (0, 0)))
    bp = jnp.pad(batch.astype(jnp.int32), (0, padn),
                 constant_values=_NB).reshape(npad, 1)
    zeros_tab = jnp.zeros((_TR, _VC), _F32)

    def call(body, ins, outs, out_specs, r=_R):
        specs = []
        for a, kind in ins:
            specs.append(_rows(kind, r) if isinstance(kind, int) else _full(a))
        return pl.pallas_call(
            body,
            grid=(npad // r,),
            in_specs=specs,
            out_specs=out_specs,
            out_shape=outs,
        )(*[a for a, _ in ins])

    def merge(parts, want_sums, want_mean):
        outs, specs = [], []
        if want_sums:
            outs.append(_sds((_TR, _VC)))
            specs.append(_full(jnp.zeros((_TR, _VC))))
        if want_mean:
            outs.append(_sds((_TR, 32)))
            specs.append(_full(jnp.zeros((_TR, 32))))
        r = pl.pallas_call(
            functools.partial(_merge_body, want_sums=want_sums,
                              want_mean=want_mean),
            grid=(1,),
            in_specs=[_full(parts)],
            out_specs=specs,
            out_shape=outs,
        )(parts)
        return r if len(r) > 1 else r[0]

    p = params
    e1o1, e1o2 = p["enc1"]["op1"], p["enc1"]["op2"]
    d0o1, d0o2 = p["dec0"]["op1"], p["dec0"]["op2"]

    ws1 = (_mlpw(p["point_encode"]) + _mlpw(p["lift"]) + _mlpw(e1o1["op_v"]))
    r1 = _R
    x0, vals1, sub = call(
        _p1_body,
        [(xp, 128), (posp, 2), (bp, 1)] + [(w, None) for w in ws1],
        [_sds((npad, 96)), _sds((npad, _VC)), _sds((npad, 1), jnp.int32)],
        [_rows(96, r1), _rows(_VC, r1), _rows(1, r1)],
        r=r1,
    )
    idx2d = sub.reshape(_NW, cpt, _CHUNK)

    def gather_pass(gat, tws):
        return call(
            _g_body,
            [(posp, 2), (bp, 1), (gat, 32)] + [(w, None) for w in tws],
            [_sds((npad, 32)), _sds((8, 65))],
            [_rows(32), _full(jnp.zeros((8, 65)))],
        )

    def ns_pass(prev, bst, ws, with_x0=False):
        r = _R
        ins = [(bp, 1), (prev, 32), (bst, None)]
        if with_x0:
            ins.append((x0, 96))
        ins += [(w, None) for w in ws]
        return call(
            functools.partial(_ns_body, nws=len(ws), with_x0=with_x0),
            ins,
            _sds((npad, _VC)),
            _rows(_VC, r),
            r=r,
        )

    def seg_roundtrip(vals, want_sums=False):
        parts = _sc_scatter(vals, idx2d, zeros_tab, cpt)
        m = merge(parts, want_sums, True)
        sums, mean = m if want_sums else (None, m)
        return sums, _sc_gather(mean, idx2d, npad, cpt)

    merged1, g1 = seg_roundtrip(vals1, want_sums=True)
    out1, bst1 = gather_pass(g1, _mlpw(e1o1["op_tgt_kernel"]))
    vals2 = ns_pass(out1, bst1, _mlpw(e1o2["op_v"]))
    _, g2 = seg_roundtrip(vals2)
    out2, bst2 = gather_pass(g2, _mlpw(e1o2["op_tgt_kernel"]))
    vals3 = ns_pass(out2, bst2, [])
    parts3 = _sc_scatter(vals3, idx2d, zeros_tab, cpt)

    mid_ws = []
    for name in ["enc2", "enc3", "enc4", "bot1", "bot2",
                 "dec4", "dec3", "dec2"]:
        for opn in ["op1", "op2"]:
            mid_ws += _mlpw(p[name][opn]["op_v"])
            mid_ws += _mlpw(p[name][opn]["op_tgt_kernel"])
    mid_ws += _mlpw(p["dec1"]["op1"]["op_v"])
    mid_ws += _mlpw(p["dec1"]["op1"]["op_tgt_kernel"])
    mid_ws += _mlpw(p["dec1"]["op2"]["op_v"])
    d1tab = call(
        _mid_body,
        [(merged1, None), (parts3, None)] + [(w, None) for w in mid_ws],
        _sds((_S, 32)),
        _full(jnp.zeros((_S, 32))),
        r=npad,
    )
    d1pad = jnp.pad(d1tab, ((0, _TR - _S), (0, 0)))
    gmid = _sc_gather(d1pad, idx2d, npad, cpt)

    out_d1, bst3 = gather_pass(gmid, _mlpw(p["dec1"]["op2"]["op_tgt_kernel"]))
    vals4 = ns_pass(out_d1, bst3, _mlpw(d0o1["op_v"]), with_x0=True)
    _, g4 = seg_roundtrip(vals4)
    out_e, bst4 = gather_pass(g4, _mlpw(d0o1["op_tgt_kernel"]))
    vals5 = ns_pass(out_e, bst4, _mlpw(d0o2["op_v"]))
    _, g5 = seg_roundtrip(vals5)
    out_f, bst5 = gather_pass(g5, _mlpw(d0o2["op_tgt_kernel"]))

    y = call(
        _p11_body,
        [(bp, 1), (out_f, 32), (bst5, None)] + [(w, None)
                                                for w in _mlpw(p["project"])],
        _sds((npad, 128)),
        _rows(128, r1),
        r=r1,
    )
    return y[:n]


# speedup vs baseline: 2.3880x; 1.0535x over previous
import functools

import jax
import jax.numpy as jnp
from jax import lax
from jax.experimental import pallas as pl
from jax.experimental.pallas import tpu as pltpu
from jax.experimental.pallas import tpu_sc as plsc

_NB = 4
_G = 16
_S = _NB * _G * _G
_R = 7168
_F32 = jnp.float32

_NC, _NS = 2, 16
_NW = _NC * _NS
_CHUNK = 112
_ALIGN = _R
_TR = 1152
_VC = 48


def _gelu(x):
    return x * 0.5 * (1.0 + lax.erf(x * 0.7071067811865476))


def _mlpw(p):
    ws = []
    for layer in p:
        ws.append(layer["W"])
        ws.append(layer["b"].reshape(1, -1))
    return ws


def _mlp(x, ws):
    n = len(ws) // 2
    for i in range(n):
        x = jnp.dot(x, ws[2 * i], preferred_element_type=_F32) + ws[2 * i + 1]
        if i < n - 1:
            x = _gelu(x)
    return x


def _dotg0(a, b):
    return lax.dot_general(a, b, (((0,), (0,)), ((), ())),
                           preferred_element_type=_F32)


def _subid(pos, batch):
    cx = jnp.clip(jnp.floor(pos[:, 0:1] * _G).astype(jnp.int32), 0, _G - 1)
    cy = jnp.clip(jnp.floor(pos[:, 1:2] * _G).astype(jnp.int32), 0, _G - 1)
    return batch * (_G * _G) + cx * _G + cy


def _bh(batch):
    ids = lax.broadcasted_iota(jnp.int32, (batch.shape[0], 8), 1)
    return (batch == ids).astype(_F32)


def _in_gelu(x, batch, bstats):
    C = x.shape[1]
    s = jnp.dot(_bh(batch), bstats, preferred_element_type=_F32)
    cnt = jnp.maximum(s[:, 2 * C:2 * C + 1], 1.0)
    mean = s[:, :C] / cnt
    var = s[:, C:2 * C] / cnt - mean * mean
    return _gelu((x - mean) / jnp.sqrt(var + 1e-5))


def _acc_init(ref):
    @pl.when(pl.program_id(0) == 0)
    def _():
        ref[...] = jnp.zeros_like(ref)


def _vals48(v, pos=None):
    r = v.shape[0]
    ones = jnp.ones((r, 1), _F32)
    mid = pos if pos is not None else jnp.zeros((r, 2), _F32)
    return jnp.concatenate([v, mid, ones, jnp.zeros((r, _VC - 35), _F32)],
                           axis=1)


def _p1_body(x_ref, pos_ref, b_ref, *rest):
    ws = [r[...] for r in rest[:-3]]
    x0_ref, vals_ref, sub_ref = rest[-3], rest[-2], rest[-1]
    pos, b = pos_ref[...], b_ref[...]
    pe = _mlp(pos, ws[0:6])
    lf = _mlp(x_ref[...], ws[6:10])
    x0 = jnp.concatenate([pe, lf], axis=1)
    x0_ref[...] = x0
    v = _mlp(x0, ws[10:14])
    vals_ref[...] = _vals48(v, pos)
    sub_ref[...] = _subid(pos, b)


def _g_body(pos_ref, b_ref, g_ref, *rest):
    ws = [r[...] for r in rest[:-2]]
    out_ref, bst_ref = rest[-2], rest[-1]
    pos, b = pos_ref[...], b_ref[...]
    out = _mlp(jnp.concatenate([pos, g_ref[...]], axis=1), ws)
    out_ref[...] = out
    ones = jnp.ones((pos.shape[0], 1), _F32)
    sb = jnp.concatenate([out, out * out, ones], axis=1)
    _acc_init(bst_ref)
    bst_ref[...] += _dotg0(_bh(b), sb)


def _ns_body(b_ref, prev_ref, bst_ref, *rest, nws, with_x0):
    k = 1 if with_x0 else 0
    x0 = rest[0][...] if with_x0 else None
    ws = [r[...] for r in rest[k:k + nws]]
    vals_ref = rest[-1]
    b = b_ref[...]
    h = _in_gelu(prev_ref[...], b, bst_ref[...])
    if with_x0:
        h = jnp.concatenate([h, x0], axis=1)
    v = _mlp(h, ws) if nws else h
    vals_ref[...] = v


def _p11_body(b_ref, prev_ref, bst_ref, *rest):
    ws = [r[...] for r in rest[:-1]]
    y_ref = rest[-1]
    h = _in_gelu(prev_ref[...], b_ref[...], bst_ref[...])
    y_ref[...] = _mlp(h, ws)


def _merge1_body(p_ref, m_ref, mean_ref):
    p = p_ref[0] + p_ref[1]
    m_ref[...] = p
    mean_ref[...] = p[:, :32] / jnp.maximum(p[:, 34:35], 1.0)


def _merge2_body(p_ref, m1_ref, mean_ref):
    p = p_ref[0] + p_ref[1]
    mean_ref[...] = p / jnp.maximum(m1_ref[:, 34:35], 1.0)


def _sc_scatter(vals, idx2d, zeros, cpt):
    mesh = plsc.VectorSubcoreMesh(core_axis_name="c", subcore_axis_name="s",
                                  num_cores=_NC, num_subcores=_NS)
    rpt = _TR // _NS

    half = cpt // 2
    rows_h = half * _CHUNK
    vc = vals.shape[1]

    @functools.partial(
        pl.kernel,
        out_type=jax.ShapeDtypeStruct((_NC, _TR, vc), _F32),
        mesh=mesh,
        scratch_types=[
            pltpu.VMEM_SHARED((_TR, vc), _F32),
            pltpu.VMEM((cpt, _CHUNK), jnp.int32),
            pltpu.VMEM((rows_h, vc), _F32),
            pltpu.SemaphoreType.DMA,
        ],
        compiler_params=pltpu.CompilerParams(use_tc_tiling_on_sc=False),
    )
    def scat(vals_hbm, idx_hbm, zeros_hbm, out_hbm, table, idx_v, buf, sem):
        cc = lax.axis_index("c")
        ss = lax.axis_index("s")
        wid = ss * _NC + cc
        pltpu.sync_copy(zeros_hbm.at[pl.ds(ss * rpt, rpt)],
                        table.at[pl.ds(ss * rpt, rpt)])
        pltpu.sync_copy(idx_hbm.at[wid], idx_v)
        plsc.subcore_barrier()
        base = wid * cpt * _CHUNK

        def phase(ph, cr):
            src = vals_hbm.at[pl.ds(base + ph * rows_h, rows_h)]
            pltpu.sync_copy(src, buf)

            def fire(j, c2):
                pltpu.async_copy(buf.at[pl.ds(j * _CHUNK, _CHUNK)],
                                 table.at[idx_v.at[ph * half + j]],
                                 sem, add=True)
                return c2

            lax.fori_loop(0, half, fire, 0)
            pltpu.make_async_copy(src, buf, sem).wait()
            return cr

        lax.fori_loop(0, 2, phase, 0)
        plsc.subcore_barrier()
        pltpu.sync_copy(table.at[pl.ds(ss * rpt, rpt)],
                        out_hbm.at[cc, pl.ds(ss * rpt, rpt)])

    return scat(vals, idx2d, zeros)


def _sc_gather(tab, idx2d, npad, cpt):
    mesh = plsc.VectorSubcoreMesh(core_axis_name="c", subcore_axis_name="s",
                                  num_cores=_NC, num_subcores=_NS)

    rows_t = cpt * _CHUNK

    @functools.partial(
        pl.kernel,
        out_type=jax.ShapeDtypeStruct((npad, 32), _F32),
        mesh=mesh,
        scratch_types=[
            pltpu.VMEM((cpt, _CHUNK), jnp.int32),
            pltpu.VMEM((rows_t, 32), _F32),
            pltpu.SemaphoreType.DMA,
        ],
        compiler_params=pltpu.CompilerParams(use_tc_tiling_on_sc=False),
    )
    def gat(tab_hbm, idx_hbm, out_hbm, idx_v, buf, sem):
        cc = lax.axis_index("c")
        ss = lax.axis_index("s")
        wid = ss * _NC + cc
        base = wid * rows_t
        pltpu.sync_copy(idx_hbm.at[wid], idx_v)

        def fire(j, cr):
            pltpu.async_copy(tab_hbm.at[idx_v.at[j]],
                             buf.at[pl.ds(j * _CHUNK, _CHUNK)], sem)
            return cr

        lax.fori_loop(0, cpt, fire, 0)
        pltpu.make_async_copy(out_hbm.at[pl.ds(base, rows_t)], buf, sem).wait()
        pltpu.sync_copy(buf, out_hbm.at[pl.ds(base, rows_t)])

    return gat(tab, idx2d)


def _mid_body(m1_ref, p3_ref, *rest):
    out_ref = rest[-1]
    loaded = iter([r[...] for r in rest[:-1]])

    def take4():
        return [next(loaded) for _ in range(4)]

    def ohm(ppos, pb, n):
        m = ppos.shape[0]
        cx = jnp.clip(jnp.floor(ppos[:, 0:1] * n).astype(jnp.int32), 0, n - 1)
        cy = jnp.clip(jnp.floor(ppos[:, 1:2] * n).astype(jnp.int32), 0, n - 1)
        sub = pb * (n * n) + cx * n + cy
        ids = lax.broadcasted_iota(jnp.int32, (m, _NB * n * n), 1)
        return (sub == ids).astype(_F32)

    def bhm(pb):
        ids = lax.broadcasted_iota(jnp.int32, (pb.shape[0], _NB), 1)
        return (pb == ids).astype(_F32)

    def dd(x, s_oh, t_oh, tpos, vws, tws):
        v = _mlp(x, vws)
        c = v.shape[1]
        ones = jnp.ones((x.shape[0], 1), _F32)
        sums = _dotg0(s_oh, jnp.concatenate([v, ones], axis=1))
        g = jnp.dot(t_oh, sums, preferred_element_type=_F32)
        mean = g[:, :c] / jnp.maximum(g[:, c:c + 1], 1.0)
        return _mlp(jnp.concatenate([tpos, mean], axis=1), tws)

    def inorm(x, bho):
        c = x.shape[1]
        ones = jnp.ones((x.shape[0], 1), _F32)
        s = _dotg0(bho, jnp.concatenate([x, x * x, ones], axis=1))
        row = jnp.dot(bho, s, preferred_element_type=_F32)
        cnt = jnp.maximum(row[:, 2 * c:2 * c + 1], 1.0)
        mean = row[:, :c] / cnt
        var = row[:, c:2 * c] / cnt - mean * mean
        return _gelu((x - mean) / jnp.sqrt(var + 1e-5))

    def blockf(x, spos, s_oh, s_bh, tpos, t_oh, t_bh):
        o = dd(x, s_oh, s_oh, spos, take4(), take4())
        o = inorm(o, s_bh)
        o = dd(o, s_oh, t_oh, tpos, take4(), take4())
        return inorm(o, t_bh)

    def pool(x, ppos, oh):
        ones = jnp.ones((x.shape[0], 1), _F32)
        ps = _dotg0(oh, jnp.concatenate([x, ppos, ones], axis=1))
        c = x.shape[1]
        cnt = jnp.maximum(ps[:, c + 2:c + 3], 1.0)
        return ps[:, :c] / cnt, ps[:, c:c + 2] / cnt

    m1 = m1_ref[...]
    m3 = p3_ref[0] + p3_ref[1]
    cnt1 = jnp.maximum(m1[:_S, 34:35], 1.0)
    p1pos = m1[:_S, 32:34] / cnt1
    p1x = m3[:_S, 0:32] / cnt1
    p1b = lax.broadcasted_iota(jnp.int32, (1024, 1), 0) // 256
    p2b = lax.broadcasted_iota(jnp.int32, (256, 1), 0) // 64
    p3b = lax.broadcasted_iota(jnp.int32, (64, 1), 0) // 16
    p4b = lax.broadcasted_iota(jnp.int32, (16, 1), 0) // 4
    bh1, bh2, bh3, bh4 = bhm(p1b), bhm(p2b), bhm(p3b), bhm(p4b)

    oh_p1_8 = ohm(p1pos, p1b, 8)
    e2 = blockf(p1x, p1pos, oh_p1_8, bh1, p1pos, oh_p1_8, bh1)
    p2x, p2pos = pool(e2, p1pos, oh_p1_8)
    oh_p2_4 = ohm(p2pos, p2b, 4)
    e3 = blockf(p2x, p2pos, oh_p2_4, bh2, p2pos, oh_p2_4, bh2)
    p3x, p3pos = pool(e3, p2pos, oh_p2_4)
    oh_p3_2 = ohm(p3pos, p3b, 2)
    e4 = blockf(p3x, p3pos, oh_p3_2, bh3, p3pos, oh_p3_2, bh3)
    p4x, p4pos = pool(e4, p3pos, oh_p3_2)
    oh_p4_1 = ohm(p4pos, p4b, 1)
    bb = blockf(p4x, p4pos, oh_p4_1, bh4, p4pos, oh_p4_1, bh4)
    bb = blockf(bb, p4pos, oh_p4_1, bh4, p4pos, oh_p4_1, bh4)
    oh_p4_2 = ohm(p4pos, p4b, 2)
    d4 = blockf(jnp.concatenate([bb, p4x], axis=1), p4pos, oh_p4_2,
                bh4, p3pos, oh_p3_2, bh3)
    oh_p3_4 = ohm(p3pos, p3b, 4)
    d3 = blockf(jnp.concatenate([d4, p3x], axis=1), p3pos, oh_p3_4,
                bh3, p2pos, oh_p2_4, bh2)
    oh_p2_8 = ohm(p2pos, p2b, 8)
    d2 = blockf(jnp.concatenate([d3, p2x], axis=1), p2pos, oh_p2_8,
                bh2, p1pos, oh_p1_8, bh1)
    oh_p1_16 = ohm(p1pos, p1b, 16)
    o = dd(jnp.concatenate([d2, p1x], axis=1), oh_p1_16, oh_p1_16,
           p1pos, take4(), take4())
    h = inorm(o, bh1)
    v = _mlp(h, take4())
    ones = jnp.ones((1024, 1), _F32)
    sums = _dotg0(oh_p1_16, jnp.concatenate([v, ones], axis=1))
    out_ref[...] = sums[:, :32] / jnp.maximum(sums[:, 32:33], 1.0)


def _rows(c, r=_R):
    return pl.BlockSpec((r, c), lambda i: (i, 0))


def _full(a):
    nd = a.ndim
    return pl.BlockSpec(a.shape, lambda i: (0,) * nd)


def _sds(shape, dt=_F32):
    return jax.ShapeDtypeStruct(shape, dt)


def kernel(x, pos, batch, params):
    n = x.shape[0]
    npad = -(-n // _ALIGN) * _ALIGN
    nt = npad // _R
    cpt = npad // (_NW * _CHUNK)
    padn = npad - n
    xp = jnp.pad(x, ((0, padn), (0, 0)))
    posp = jnp.pad(pos, ((0, padn), (0, 0)))
    bp = jnp.pad(batch.astype(jnp.int32), (0, padn),
                 constant_values=_NB).reshape(npad, 1)
    zeros48 = jnp.zeros((_TR, _VC), _F32)
    zeros32 = jnp.zeros((_TR, 32), _F32)

    def call(body, ins, outs, out_specs, r=_R):
        specs = []
        for a, kind in ins:
            specs.append(_rows(kind, r) if isinstance(kind, int) else _full(a))
        return pl.pallas_call(
            body,
            grid=(npad // r,),
            in_specs=specs,
            out_specs=out_specs,
            out_shape=outs,
        )(*[a for a, _ in ins])

    mean_spec = _full(jnp.zeros((_TR, 32)))

    def merge1(parts):
        return pl.pallas_call(
            _merge1_body,
            grid=(1,),
            in_specs=[_full(parts)],
            out_specs=[_full(jnp.zeros((_TR, _VC))), mean_spec],
            out_shape=[_sds((_TR, _VC)), _sds((_TR, 32))],
        )(parts)

    def merge2(parts, m1):
        return pl.pallas_call(
            _merge2_body,
            grid=(1,),
            in_specs=[_full(parts), _full(m1)],
            out_specs=mean_spec,
            out_shape=_sds((_TR, 32)),
        )(parts, m1)

    p = params
    e1o1, e1o2 = p["enc1"]["op1"], p["enc1"]["op2"]
    d0o1, d0o2 = p["dec0"]["op1"], p["dec0"]["op2"]

    ws1 = (_mlpw(p["point_encode"]) + _mlpw(p["lift"]) + _mlpw(e1o1["op_v"]))
    r1 = _R
    x0, vals1, sub = call(
        _p1_body,
        [(xp, 128), (posp, 2), (bp, 1)] + [(w, None) for w in ws1],
        [_sds((npad, 96)), _sds((npad, _VC)), _sds((npad, 1), jnp.int32)],
        [_rows(96, r1), _rows(_VC, r1), _rows(1, r1)],
        r=r1,
    )
    idx2d = sub.reshape(_NW, cpt, _CHUNK)

    def gather_pass(gat, tws):
        return call(
            _g_body,
            [(posp, 2), (bp, 1), (gat, 32)] + [(w, None) for w in tws],
            [_sds((npad, 32)), _sds((8, 65))],
            [_rows(32), _full(jnp.zeros((8, 65)))],
        )

    def ns_pass(prev, bst, ws, with_x0=False):
        r = _R
        ins = [(bp, 1), (prev, 32), (bst, None)]
        if with_x0:
            ins.append((x0, 96))
        ins += [(w, None) for w in ws]
        return call(
            functools.partial(_ns_body, nws=len(ws), with_x0=with_x0),
            ins,
            _sds((npad, 32)),
            _rows(32, r),
            r=r,
        )

    parts1 = _sc_scatter(vals1, idx2d, zeros48, cpt)
    merged1, mean1 = merge1(parts1)
    g1 = _sc_gather(mean1, idx2d, npad, cpt)

    def seg_roundtrip(vals):
        parts = _sc_scatter(vals, idx2d, zeros32, cpt)
        return _sc_gather(merge2(parts, merged1), idx2d, npad, cpt)

    out1, bst1 = gather_pass(g1, _mlpw(e1o1["op_tgt_kernel"]))
    vals2 = ns_pass(out1, bst1, _mlpw(e1o2["op_v"]))
    g2 = seg_roundtrip(vals2)
    out2, bst2 = gather_pass(g2, _mlpw(e1o2["op_tgt_kernel"]))
    vals3 = ns_pass(out2, bst2, [])
    parts3 = _sc_scatter(vals3, idx2d, zeros32, cpt)

    mid_ws = []
    for name in ["enc2", "enc3", "enc4", "bot1", "bot2",
                 "dec4", "dec3", "dec2"]:
        for opn in ["op1", "op2"]:
            mid_ws += _mlpw(p[name][opn]["op_v"])
            mid_ws += _mlpw(p[name][opn]["op_tgt_kernel"])
    mid_ws += _mlpw(p["dec1"]["op1"]["op_v"])
    mid_ws += _mlpw(p["dec1"]["op1"]["op_tgt_kernel"])
    mid_ws += _mlpw(p["dec1"]["op2"]["op_v"])
    d1tab = call(
        _mid_body,
        [(merged1, None), (parts3, None)] + [(w, None) for w in mid_ws],
        _sds((_S, 32)),
        _full(jnp.zeros((_S, 32))),
        r=npad,
    )
    d1pad = jnp.pad(d1tab, ((0, _TR - _S), (0, 0)))
    gmid = _sc_gather(d1pad, idx2d, npad, cpt)

    out_d1, bst3 = gather_pass(gmid, _mlpw(p["dec1"]["op2"]["op_tgt_kernel"]))
    vals4 = ns_pass(out_d1, bst3, _mlpw(d0o1["op_v"]), with_x0=True)
    g4 = seg_roundtrip(vals4)
    out_e, bst4 = gather_pass(g4, _mlpw(d0o1["op_tgt_kernel"]))
    vals5 = ns_pass(out_e, bst4, _mlpw(d0o2["op_v"]))
    g5 = seg_roundtrip(vals5)
    out_f, bst5 = gather_pass(g5, _mlpw(d0o2["op_tgt_kernel"]))

    y = call(
        _p11_body,
        [(bp, 1), (out_f, 32), (bst5, None)] + [(w, None)
                                                for w in _mlpw(p["project"])],
        _sds((npad, 128)),
        _rows(128, r1),
        r=r1,
    )
    return y[:n]
